# baseline (speedup 1.0000x reference)
import functools
import jax
import jax.numpy as jnp
from jax import lax
from jax.experimental import pallas as pl
from jax.experimental.pallas import tpu as pltpu
from jax.experimental.pallas import tpu_sc as plsc

F32 = jnp.float32
I32 = jnp.int32

N = 10000
E = 160000
G = 16
D_OUT = 512
NP = 10240
E2 = E + N
EPT = 10656
E2P = 16 * EPT
NGRP = EPT // 16
CPT = E // 16
CGRP = CPT // 16
BAND = NP // 16
NB4 = 4
FB = 128
RING = 4

@functools.lru_cache(maxsize=None)
def _get_mesh():
    return plsc.VectorSubcoreMesh(core_axis_name="c", subcore_axis_name="s")


def _sc_loop_mean_body(dst_hbm, s1_hbm, s2_hbm, l1_hbm, l2_hbm,
                       dst_v, s1_v, s2_v, cnt_v, su1_v, su2_v,
                       red_v, a_v, b_v, c_v, part):
    c = lax.axis_index("c")
    s = lax.axis_index("s")
    pltpu.sync_copy(dst_hbm.at[s], dst_v)
    pltpu.sync_copy(s1_hbm.at[s], s1_v)
    pltpu.sync_copy(s2_hbm.at[s], s2_v)

    zero16 = jnp.zeros((16,), F32)

    def zbody(i, _):
        cnt_v[pl.ds(i * 16, 16)] = zero16
        su1_v[pl.ds(i * 16, 16)] = zero16
        su2_v[pl.ds(i * 16, 16)] = zero16
        return 0
    lax.fori_loop(0, NP // 16, zbody, 0)

    one16 = jnp.full((16,), 1.0, F32)

    def ebody(g, _):
        dg = dst_v[g]
        plsc.addupdate_scatter(cnt_v, [dg], one16)
        plsc.addupdate_scatter(su1_v, [dg], s1_v[g])
        plsc.addupdate_scatter(su2_v, [dg], s2_v[g])
        return 0
    lax.fori_loop(0, CGRP, ebody, 0)

    pltpu.sync_copy(cnt_v, part.at[0, s])
    pltpu.sync_copy(su1_v, part.at[1, s])
    pltpu.sync_copy(su2_v, part.at[2, s])
    plsc.subcore_barrier()

    def _reduce(tab, outbuf):
        pltpu.sync_copy(part.at[tab, :, pl.ds(s * BAND, BAND)], red_v)

        def rbody(j, _):
            acc = jnp.zeros((16,), F32)
            for t in range(16):
                acc = acc + red_v[t, pl.ds(j * 16, 16)]
            outbuf[pl.ds(j * 16, 16)] = acc
            return 0
        lax.fori_loop(0, BAND // 16, rbody, 0)

    _reduce(0, c_v)
    _reduce(1, a_v)
    _reduce(2, b_v)

    def dbody(j, _):
        cc = jnp.maximum(c_v[pl.ds(j * 16, 16)], 1.0)
        a_v[pl.ds(j * 16, 16)] = a_v[pl.ds(j * 16, 16)] / cc
        b_v[pl.ds(j * 16, 16)] = b_v[pl.ds(j * 16, 16)] / cc
        return 0
    lax.fori_loop(0, BAND // 16, dbody, 0)

    @pl.when(c == 0)
    def _():
        pltpu.sync_copy(a_v, l1_hbm.at[pl.ds(s * BAND, BAND)])
        pltpu.sync_copy(b_v, l2_hbm.at[pl.ds(s * BAND, BAND)])


@functools.lru_cache(maxsize=None)
def _get_sc_loop_mean():
    return pl.kernel(
        _sc_loop_mean_body,
        out_type=(jax.ShapeDtypeStruct((NP,), F32),
                  jax.ShapeDtypeStruct((NP,), F32)),
        mesh=_get_mesh(),
        compiler_params=pltpu.CompilerParams(needs_layout_passes=False,
                                             use_tc_tiling_on_sc=False),
        scratch_types=[
        pltpu.VMEM((CGRP, 16), I32),
        pltpu.VMEM((CGRP, 16), F32),
        pltpu.VMEM((CGRP, 16), F32),
        pltpu.VMEM((NP,), F32),
        pltpu.VMEM((NP,), F32),
        pltpu.VMEM((NP,), F32),
        pltpu.VMEM((16, BAND), F32),
        pltpu.VMEM((BAND,), F32),
        pltpu.VMEM((BAND,), F32),
            pltpu.VMEM((BAND,), F32),
            pltpu.VMEM_SHARED((3, 16, NP), F32),
        ],
    )


def _sc_edge_w_body(avs_hbm, avd_hbm, src_hbm, dst_hbm, se_hbm,
                    w_hbm, den_hbm,
                    avs_v, avd_v, src_v, dst_v, se_v, w_v, den_v,
                    red_v, dout_v, denp):
    c = lax.axis_index("c")
    s = lax.axis_index("s")

    pltpu.sync_copy(avs_hbm, avs_v)
    pltpu.sync_copy(avd_hbm, avd_v)
    pltpu.sync_copy(src_hbm.at[s], src_v)
    pltpu.sync_copy(dst_hbm.at[s], dst_v)
    pltpu.sync_copy(se_hbm.at[s], se_v)

    zero16 = jnp.zeros((16,), F32)

    def zbody(i, _):
        den_v[pl.ds(i * 16, 16)] = zero16
        return 0
    lax.fori_loop(0, NP // 16, zbody, 0)

    def p1body(g, _):
        sg = src_v[g]
        dg = dst_v[g]
        a = (plsc.load_gather(avs_v, [sg]) + plsc.load_gather(avd_v, [dg])
             + se_v[g])
        a = jnp.where(a > 0, a, 0.2 * a)
        w = jnp.exp(a)
        w_v[g] = w
        plsc.addupdate_scatter(den_v, [dg], w)
        return 0
    lax.fori_loop(0, NGRP, p1body, 0)

    @pl.when(c == 0)
    def _():
        pltpu.sync_copy(w_v, w_hbm.at[s])

    pltpu.sync_copy(den_v, denp.at[s])
    plsc.subcore_barrier()

    pltpu.sync_copy(denp.at[:, pl.ds(s * BAND, BAND)], red_v)

    def rbody(j, _):
        acc = jnp.zeros((16,), F32)
        for t in range(16):
            acc = acc + red_v[t, pl.ds(j * 16, 16)]
        dout_v[pl.ds(j * 16, 16)] = acc
        return 0
    lax.fori_loop(0, BAND // 16, rbody, 0)

    @pl.when(c == 0)
    def _():
        pltpu.sync_copy(dout_v, den_hbm.at[pl.ds(s * BAND, BAND)])


@functools.lru_cache(maxsize=None)
def _get_sc_edge_w():
    return pl.kernel(
        _sc_edge_w_body,
        out_type=(jax.ShapeDtypeStruct((16, NGRP, 16), F32),
                  jax.ShapeDtypeStruct((NP,), F32)),
        mesh=_get_mesh(),
        compiler_params=pltpu.CompilerParams(needs_layout_passes=False,
                                             use_tc_tiling_on_sc=False),
        scratch_types=[
            pltpu.VMEM((NP,), F32),
            pltpu.VMEM((NP,), F32),
            pltpu.VMEM((NGRP, 16), I32),
            pltpu.VMEM((NGRP, 16), I32),
            pltpu.VMEM((NGRP, 16), F32),
            pltpu.VMEM((NGRP, 16), F32),
            pltpu.VMEM((NP,), F32),
            pltpu.VMEM((16, BAND), F32),
            pltpu.VMEM((BAND,), F32),
            pltpu.VMEM_SHARED((16, NP), F32),
        ],
    )


def _sc_agg_body(hw_hbm, src_hbm, dst_hbm, w_hbm, zer_hbm, out_hbm,
                 src_v, dst_v, w_v, gring, sring, gsem, ssem, table):
    c = lax.axis_index("c")
    s = lax.axis_index("s")

    pltpu.sync_copy(src_hbm.at[s], src_v)
    pltpu.sync_copy(dst_hbm.at[s], dst_v)
    pltpu.sync_copy(w_hbm.at[s], w_v)

    for bi in range(2):
        b = bi * 2 + c
        base = b * NP

        pltpu.sync_copy(zer_hbm, table.at[pl.ds(s * BAND, BAND)])
        plsc.subcore_barrier()

        def gstart(g, slot):
            idx = src_v[g] + base
            pltpu.async_copy(hw_hbm.at[idx], gring.at[slot], gsem.at[slot])

        for slot in range(RING):
            gstart(slot, slot)

        def process(g, slot, do_swait):
            gb = gring.at[slot]
            sb = sring.at[slot]
            pltpu.make_async_copy(hw_hbm.at[pl.ds(0, 16)], gb,
                                  gsem.at[slot]).wait()
            if do_swait:
                pltpu.make_async_copy(sb, table.at[pl.ds(0, 16)],
                                      ssem.at[slot]).wait()
            wg = w_v[g]
            for r in range(16):
                wr = wg[r]
                for k in range(FB // 16):
                    sb[r, pl.ds(k * 16, 16)] = gb[r, pl.ds(k * 16, 16)] * wr
            pltpu.async_copy(sb, table.at[dst_v[g]], ssem.at[slot], add=True)

        def pair(g2, do_swait, do_gstart):
            g = g2 * 2
            s0 = jnp.bitwise_and(g, RING - 1)
            process(g, s0, do_swait)
            if do_gstart:
                gstart(g + RING, s0)
            s1 = jnp.bitwise_and(g + 1, RING - 1)
            process(g + 1, s1, do_swait)
            if do_gstart:
                gstart(g + 1 + RING, s1)

        def warm(i, _):
            pair(i, False, True)
            return 0

        def steady(i, _):
            pair(i, True, True)
            return 0

        def cool(i, _):
            pair(i, True, False)
            return 0
        lax.fori_loop(0, RING // 2, warm, 0)
        lax.fori_loop(RING // 2, NGRP // 2 - RING // 2, steady, 0)
        lax.fori_loop(NGRP // 2 - RING // 2, NGRP // 2, cool, 0)

        for slot in range(RING):
            pltpu.make_async_copy(sring.at[slot], table.at[pl.ds(0, 16)],
                                  ssem.at[slot]).wait()
        plsc.subcore_barrier()

        pltpu.sync_copy(table.at[pl.ds(s * BAND, BAND)],
                        out_hbm.at[pl.ds(base + s * BAND, BAND)])
        plsc.subcore_barrier()


@functools.lru_cache(maxsize=None)
def _get_sc_agg():
    return pl.kernel(
        _sc_agg_body,
        out_type=jax.ShapeDtypeStruct((NB4 * NP, FB), F32),
        mesh=_get_mesh(),
        compiler_params=pltpu.CompilerParams(needs_layout_passes=False,
                                             use_tc_tiling_on_sc=False),
        scratch_types=[
            pltpu.VMEM((NGRP, 16), I32),
            pltpu.VMEM((NGRP, 16), I32),
            pltpu.VMEM((NGRP, 16), F32),
            pltpu.VMEM((RING, 16, FB), F32),
            pltpu.VMEM((RING, 16, FB), F32),
            pltpu.SemaphoreType.DMA((RING,)),
            pltpu.SemaphoreType.DMA((RING,)),
            pltpu.VMEM_SHARED((NP, FB), F32),
        ],
    )


_BR = 512
_NBR = NP // _BR


def _lin_body(h_ref, w_ref, b_ref, as_ref, ad_ref, hw_ref, avs_ref, avd_ref):
    b = pl.program_id(1)
    hwb = jnp.dot(h_ref[...], w_ref[...], preferred_element_type=F32)
    hwb = hwb + b_ref[...]
    hw_ref[...] = hwb
    pa = jnp.dot(hwb, as_ref[...].reshape(FB), preferred_element_type=F32)
    pd = jnp.dot(hwb, ad_ref[...].reshape(FB), preferred_element_type=F32)

    @pl.when(b == 0)
    def _():
        avs_ref[...] = jnp.zeros((1, 1, _BR), F32)
        avd_ref[...] = jnp.zeros((1, 1, _BR), F32)
    avs_ref[...] += pa.reshape(1, 1, _BR)
    avd_ref[...] += pd.reshape(1, 1, _BR)


def _lin(h, W, bias, asv, adv):
    K = h.shape[1]
    hw, avs, avd = pl.pallas_call(
        _lin_body,
        grid=(_NBR, NB4),
        in_specs=[
            pl.BlockSpec((_BR, K), lambda i, b: (i, 0)),
            pl.BlockSpec((K, FB), lambda i, b: (0, b)),
            pl.BlockSpec((1, FB), lambda i, b: (0, b)),
            pl.BlockSpec((1, FB), lambda i, b: (0, b)),
            pl.BlockSpec((1, FB), lambda i, b: (0, b)),
        ],
        out_specs=[
            pl.BlockSpec((_BR, FB), lambda i, b: (b * _NBR + i, 0)),
            pl.BlockSpec((1, 1, _BR), lambda i, b: (i, 0, 0)),
            pl.BlockSpec((1, 1, _BR), lambda i, b: (i, 0, 0)),
        ],
        out_shape=[
            jax.ShapeDtypeStruct((NB4 * NP, FB), F32),
            jax.ShapeDtypeStruct((_NBR, 1, _BR), F32),
            jax.ShapeDtypeStruct((_NBR, 1, _BR), F32),
        ],
    )(h, W, bias.reshape(1, D_OUT), asv.reshape(1, D_OUT),
      adv.reshape(1, D_OUT))
    return hw, avs.reshape(NP), avd.reshape(NP)


def _act_body(o0, o1, o2, o3, den_ref, b_ref, h_ref):
    cat = jnp.concatenate([o0[...], o1[...], o2[...], o3[...]], axis=1)
    den = den_ref[...].reshape(_BR, 1)
    h_ref[...] = jnp.maximum(cat / (den + 1e-16) + b_ref[...], 0.0)


def _act(outflat, den, bias):
    specs = [pl.BlockSpec((_BR, FB), (lambda j: (lambda i: (j * _NBR + i, 0)))(j))
             for j in range(4)]
    return pl.pallas_call(
        _act_body,
        grid=(_NBR,),
        in_specs=specs + [
            pl.BlockSpec((1, 1, _BR), lambda i: (i, 0, 0)),
            pl.BlockSpec((1, D_OUT), lambda i: (0, 0)),
        ],
        out_specs=pl.BlockSpec((_BR, D_OUT), lambda i: (i, 0)),
        out_shape=jax.ShapeDtypeStruct((NP, D_OUT), F32),
    )(outflat, outflat, outflat, outflat, den.reshape(_NBR, 1, _BR),
      bias.reshape(1, D_OUT))


def _edge_scalar_body(ea_ref, u_ref, c_ref, s1_ref, s2_ref):
    sblk = jnp.dot(ea_ref[...], u_ref[...], preferred_element_type=F32)
    sblk = sblk + c_ref[...]
    s1_ref[...] = sblk[:, 0].reshape(1, 1, -1)
    s2_ref[...] = sblk[:, 1].reshape(1, 1, -1)


def _edge_scalars(edge_attr, U, c2):
    BE = 2000
    nb = E // BE
    D = edge_attr.shape[1]
    s1, s2 = pl.pallas_call(
        _edge_scalar_body,
        grid=(nb,),
        in_specs=[
            pl.BlockSpec((BE, D), lambda i: (i, 0)),
            pl.BlockSpec((D, 2), lambda i: (0, 0)),
            pl.BlockSpec((1, 2), lambda i: (0, 0)),
        ],
        out_specs=[
            pl.BlockSpec((1, 1, BE), lambda i: (i, 0, 0)),
            pl.BlockSpec((1, 1, BE), lambda i: (i, 0, 0)),
        ],
        out_shape=[
            jax.ShapeDtypeStruct((nb, 1, BE), F32),
            jax.ShapeDtypeStruct((nb, 1, BE), F32),
        ],
    )(edge_attr, U, c2.reshape(1, 2))
    return s1.reshape(E), s2.reshape(E)


def _pool_body(o0, o1, o2, o3, den_ref, b_ref, wp_ref, bp_ref, batch_ref,
               sums_ref):
    i = pl.program_id(0)
    cat = jnp.concatenate([o0[...], o1[...], o2[...], o3[...]], axis=1)
    den = den_ref[...].reshape(_BR, 1)
    hp = cat / (den + 1e-16) + b_ref[...]
    sc = jax.nn.sigmoid(jnp.dot(hp, wp_ref[...],
                                preferred_element_type=F32) + bp_ref[0, 0])
    xw = hp * sc
    bv = batch_ref[...].reshape(_BR, 1)
    oh = (bv == lax.broadcasted_iota(I32, (_BR, G), 1)).astype(F32)
    contrib = lax.dot_general(oh, xw, (((0,), (0,)), ((), ())),
                              preferred_element_type=F32)

    @pl.when(i == 0)
    def _():
        sums_ref[...] = jnp.zeros((G, D_OUT), F32)
    sums_ref[...] += contrib


def _pool(outflat, den, bias, wproj, bproj, batch3):
    specs = [pl.BlockSpec((_BR, FB), (lambda j: (lambda i: (j * _NBR + i, 0)))(j))
             for j in range(4)]
    return pl.pallas_call(
        _pool_body,
        grid=(_NBR,),
        in_specs=specs + [
            pl.BlockSpec((1, 1, _BR), lambda i: (i, 0, 0)),
            pl.BlockSpec((1, D_OUT), lambda i: (0, 0)),
            pl.BlockSpec((D_OUT, 1), lambda i: (0, 0)),
            pl.BlockSpec((1, 1), lambda i: (0, 0)),
            pl.BlockSpec((1, 1, _BR), lambda i: (i, 0, 0)),
        ],
        out_specs=pl.BlockSpec((G, D_OUT), lambda i: (0, 0)),
        out_shape=jax.ShapeDtypeStruct((G, D_OUT), F32),
    )(outflat, outflat, outflat, outflat, den.reshape(_NBR, 1, _BR),
      bias.reshape(1, D_OUT), wproj, bproj.reshape(1, 1), batch3)


def _fin_body(s1_ref, s2_ref, batch_ref, out_ref):
    bm = batch_ref[...].reshape(_NBR, _BR)
    acc = jnp.zeros((G,), F32)
    for r in range(_NBR):
        oh = (bm[r].reshape(_BR, 1)
              == lax.broadcasted_iota(I32, (_BR, G), 1)).astype(F32)
        acc = acc + jnp.sum(oh, axis=0)
    cnt = jnp.maximum(acc, 1.0).reshape(G, 1)
    out_ref[...] = jnp.concatenate(
        [s1_ref[...] / cnt, s2_ref[...] / cnt], axis=1)


def _fin(sums1, sums2, batch3):
    return pl.pallas_call(
        _fin_body,
        grid=(1,),
        in_specs=[
            pl.BlockSpec((G, D_OUT), lambda i: (0, 0)),
            pl.BlockSpec((G, D_OUT), lambda i: (0, 0)),
            pl.BlockSpec((_NBR, 1, _BR), lambda i: (0, 0, 0)),
        ],
        out_specs=pl.BlockSpec((G, 2 * D_OUT), lambda i: (0, 0)),
        out_shape=jax.ShapeDtypeStruct((G, 2 * D_OUT), F32),
    )(sums1, sums2, batch3)


def kernel(x, edge_index, edge_attr, batch, Wnp, bnp, Wep, bep,
           W1, as1, ad1, b1, We1, ae1, W2, as2, ad2, b2, We2, ae2,
           Wp1, asp1, adp1, bp1, wproj1, bproj1,
           Wp2, asp2, adp2, bp2, wproj2, bproj2):
    src = edge_index[0]
    dst = edge_index[1]

    v1 = We1 @ ae1
    v2 = We2 @ ae2
    U = jnp.stack([Wep @ v1, Wep @ v2], axis=1)
    c2 = jnp.stack([bep @ v1, bep @ v2])
    M1 = Wnp @ W1
    m1b = bnp @ W1

    loop_ids = jnp.arange(N, dtype=I32)
    pad_i = jnp.zeros((E2P - E2,), I32)
    srcg = jnp.concatenate([src, loop_ids, pad_i]).reshape(16, NGRP, 16)
    dstg = jnp.concatenate([dst, loop_ids, pad_i]).reshape(16, NGRP, 16)
    pad_f = jnp.full((E2P - E2,), -1e30, F32)
    dst_r = dst.reshape(16, CGRP, 16)
    zer = jnp.zeros((BAND, FB), F32)
    zeros512 = jnp.zeros((D_OUT,), F32)
    x_pad = jnp.concatenate([x, jnp.zeros((NP - N, x.shape[1]), F32)], axis=0)
    batch3 = jnp.concatenate(
        [batch, jnp.full((NP - N,), -1, I32)]).reshape(_NBR, 1, _BR)

    s1, s2 = _edge_scalars(edge_attr, U, c2)
    l1, l2 = _get_sc_loop_mean()(dst_r, s1.reshape(16, CGRP, 16),
                                 s2.reshape(16, CGRP, 16))
    se1 = jnp.concatenate([s1, l1[:N], pad_f]).reshape(16, NGRP, 16)
    se2 = jnp.concatenate([s2, l2[:N], pad_f]).reshape(16, NGRP, 16)
    se0 = jnp.concatenate([jnp.zeros((E2,), F32), pad_f]).reshape(16, NGRP, 16)

    hw1, avs1, avd1 = _lin(x_pad, M1, m1b, as1, ad1)
    w_out1, den1 = _get_sc_edge_w()(avs1, avd1, srcg, dstg, se1)
    out1 = _get_sc_agg()(hw1, srcg, dstg, w_out1, zer)
    h1 = _act(out1, den1, b1)

    hw2, avs2, avd2 = _lin(h1, W2, zeros512, as2, ad2)
    w_out2, den2 = _get_sc_edge_w()(avs2, avd2, srcg, dstg, se2)
    out2 = _get_sc_agg()(hw2, srcg, dstg, w_out2, zer)
    h2 = _act(out2, den2, b2)

    hwp1, avsp1, avdp1 = _lin(h2, Wp1, zeros512, asp1, adp1)
    w_outp1, denp1 = _get_sc_edge_w()(avsp1, avdp1, srcg, dstg, se0)
    outp1 = _get_sc_agg()(hwp1, srcg, dstg, w_outp1, zer)
    sums1 = _pool(outp1, denp1, bp1, wproj1, bproj1, batch3)

    hwp2, avsp2, avdp2 = _lin(h2, Wp2, zeros512, asp2, adp2)
    w_outp2, denp2 = _get_sc_edge_w()(avsp2, avdp2, srcg, dstg, se0)
    outp2 = _get_sc_agg()(hwp2, srcg, dstg, w_outp2, zer)
    sums2 = _pool(outp2, denp2, bp2, wproj2, bproj2, batch3)

    return _fin(sums1, sums2, batch3)


# speedup vs baseline: 1.0086x; 1.0086x over previous
import functools
import jax
import jax.numpy as jnp
from jax import lax
from jax.experimental import pallas as pl
from jax.experimental.pallas import tpu as pltpu
from jax.experimental.pallas import tpu_sc as plsc

F32 = jnp.float32
I32 = jnp.int32

N = 10000
E = 160000
G = 16
D_OUT = 512
NP = 10240
E2 = E + N
EPT = 10656
E2P = 16 * EPT
NGRP = EPT // 16
CPT = E // 16
CGRP = CPT // 16
BAND = NP // 16
NB4 = 4
FB = 128
RING = 4

@functools.lru_cache(maxsize=None)
def _get_mesh():
    return plsc.VectorSubcoreMesh(core_axis_name="c", subcore_axis_name="s")


def _sc_loop_mean_body(dst_hbm, s1_hbm, s2_hbm, l1_hbm, l2_hbm,
                       dst_v, s1_v, s2_v, cnt_v, su1_v, su2_v,
                       red_v, a_v, b_v, c_v, part):
    c = lax.axis_index("c")
    s = lax.axis_index("s")
    pltpu.sync_copy(dst_hbm.at[s], dst_v)
    pltpu.sync_copy(s1_hbm.at[s], s1_v)
    pltpu.sync_copy(s2_hbm.at[s], s2_v)

    zero16 = jnp.zeros((16,), F32)

    def zbody(i, _):
        cnt_v[pl.ds(i * 16, 16)] = zero16
        su1_v[pl.ds(i * 16, 16)] = zero16
        su2_v[pl.ds(i * 16, 16)] = zero16
        return 0
    lax.fori_loop(0, NP // 16, zbody, 0)

    one16 = jnp.full((16,), 1.0, F32)

    def ebody(g, _):
        dg = dst_v[g]
        plsc.addupdate_scatter(cnt_v, [dg], one16)
        plsc.addupdate_scatter(su1_v, [dg], s1_v[g])
        plsc.addupdate_scatter(su2_v, [dg], s2_v[g])
        return 0
    lax.fori_loop(0, CGRP, ebody, 0)

    pltpu.sync_copy(cnt_v, part.at[0, s])
    pltpu.sync_copy(su1_v, part.at[1, s])
    pltpu.sync_copy(su2_v, part.at[2, s])
    plsc.subcore_barrier()

    def _reduce(tab, outbuf):
        pltpu.sync_copy(part.at[tab, :, pl.ds(s * BAND, BAND)], red_v)

        def rbody(j, _):
            acc = jnp.zeros((16,), F32)
            for t in range(16):
                acc = acc + red_v[t, pl.ds(j * 16, 16)]
            outbuf[pl.ds(j * 16, 16)] = acc
            return 0
        lax.fori_loop(0, BAND // 16, rbody, 0)

    _reduce(0, c_v)
    _reduce(1, a_v)
    _reduce(2, b_v)

    def dbody(j, _):
        cc = jnp.maximum(c_v[pl.ds(j * 16, 16)], 1.0)
        a_v[pl.ds(j * 16, 16)] = a_v[pl.ds(j * 16, 16)] / cc
        b_v[pl.ds(j * 16, 16)] = b_v[pl.ds(j * 16, 16)] / cc
        return 0
    lax.fori_loop(0, BAND // 16, dbody, 0)

    @pl.when(c == 0)
    def _():
        pltpu.sync_copy(a_v, l1_hbm.at[pl.ds(s * BAND, BAND)])
        pltpu.sync_copy(b_v, l2_hbm.at[pl.ds(s * BAND, BAND)])


@functools.lru_cache(maxsize=None)
def _get_sc_loop_mean():
    return pl.kernel(
        _sc_loop_mean_body,
        out_type=(jax.ShapeDtypeStruct((NP,), F32),
                  jax.ShapeDtypeStruct((NP,), F32)),
        mesh=_get_mesh(),
        compiler_params=pltpu.CompilerParams(needs_layout_passes=False,
                                             use_tc_tiling_on_sc=False),
        scratch_types=[
        pltpu.VMEM((CGRP, 16), I32),
        pltpu.VMEM((CGRP, 16), F32),
        pltpu.VMEM((CGRP, 16), F32),
        pltpu.VMEM((NP,), F32),
        pltpu.VMEM((NP,), F32),
        pltpu.VMEM((NP,), F32),
        pltpu.VMEM((16, BAND), F32),
        pltpu.VMEM((BAND,), F32),
        pltpu.VMEM((BAND,), F32),
            pltpu.VMEM((BAND,), F32),
            pltpu.VMEM_SHARED((3, 16, NP), F32),
        ],
    )


def _sc_edge_w_body(avs_hbm, avd_hbm, src_hbm, dst_hbm, se_hbm,
                    w_hbm, den_hbm,
                    avs_v, avd_v, src_v, dst_v, se_v, w_v, den_v,
                    red_v, dout_v, denp):
    c = lax.axis_index("c")
    s = lax.axis_index("s")

    pltpu.sync_copy(avs_hbm, avs_v)
    pltpu.sync_copy(avd_hbm, avd_v)
    pltpu.sync_copy(src_hbm.at[s], src_v)
    pltpu.sync_copy(dst_hbm.at[s], dst_v)
    pltpu.sync_copy(se_hbm.at[s], se_v)

    zero16 = jnp.zeros((16,), F32)

    def zbody(i, _):
        den_v[pl.ds(i * 16, 16)] = zero16
        return 0
    lax.fori_loop(0, NP // 16, zbody, 0)

    def p1body(g, _):
        sg = src_v[g]
        dg = dst_v[g]
        a = (plsc.load_gather(avs_v, [sg]) + plsc.load_gather(avd_v, [dg])
             + se_v[g])
        a = jnp.where(a > 0, a, 0.2 * a)
        w = jnp.exp(a)
        w_v[g] = w
        plsc.addupdate_scatter(den_v, [dg], w)
        return 0
    lax.fori_loop(0, NGRP, p1body, 0)

    @pl.when(c == 0)
    def _():
        pltpu.sync_copy(w_v, w_hbm.at[s])

    pltpu.sync_copy(den_v, denp.at[s])
    plsc.subcore_barrier()

    pltpu.sync_copy(denp.at[:, pl.ds(s * BAND, BAND)], red_v)

    def rbody(j, _):
        acc = jnp.zeros((16,), F32)
        for t in range(16):
            acc = acc + red_v[t, pl.ds(j * 16, 16)]
        dout_v[pl.ds(j * 16, 16)] = acc
        return 0
    lax.fori_loop(0, BAND // 16, rbody, 0)

    @pl.when(c == 0)
    def _():
        pltpu.sync_copy(dout_v, den_hbm.at[pl.ds(s * BAND, BAND)])


@functools.lru_cache(maxsize=None)
def _get_sc_edge_w():
    return pl.kernel(
        _sc_edge_w_body,
        out_type=(jax.ShapeDtypeStruct((16, NGRP, 16), F32),
                  jax.ShapeDtypeStruct((NP,), F32)),
        mesh=_get_mesh(),
        compiler_params=pltpu.CompilerParams(needs_layout_passes=False,
                                             use_tc_tiling_on_sc=False),
        scratch_types=[
            pltpu.VMEM((NP,), F32),
            pltpu.VMEM((NP,), F32),
            pltpu.VMEM((NGRP, 16), I32),
            pltpu.VMEM((NGRP, 16), I32),
            pltpu.VMEM((NGRP, 16), F32),
            pltpu.VMEM((NGRP, 16), F32),
            pltpu.VMEM((NP,), F32),
            pltpu.VMEM((16, BAND), F32),
            pltpu.VMEM((BAND,), F32),
            pltpu.VMEM_SHARED((16, NP), F32),
        ],
    )


def _sc_agg_body(hw_hbm, src_hbm, dst_hbm, w_hbm, zer_hbm, out_hbm,
                 src_v, dst_v, w_v, gring, sring, gsem, ssem, table):
    c = lax.axis_index("c")
    s = lax.axis_index("s")

    pltpu.sync_copy(src_hbm.at[s], src_v)
    pltpu.sync_copy(dst_hbm.at[s], dst_v)
    pltpu.sync_copy(w_hbm.at[s], w_v)

    for bi in range(2):
        b = bi * 2 + c
        base = b * NP

        pltpu.sync_copy(zer_hbm, table.at[pl.ds(s * BAND, BAND)])
        plsc.subcore_barrier()

        def gstart(g, slot):
            idx = src_v[g] + base
            pltpu.async_copy(hw_hbm.at[idx], gring.at[slot], gsem.at[slot])

        for slot in range(RING):
            gstart(slot, slot)

        def mbody(g, _):
            slot = lax.rem(g, RING)
            gb = gring.at[slot]
            sb = sring.at[slot]
            pltpu.make_async_copy(hw_hbm.at[pl.ds(0, 16)], gb,
                                  gsem.at[slot]).wait()

            @pl.when(g >= RING)
            def _():
                pltpu.make_async_copy(sb, table.at[pl.ds(0, 16)],
                                      ssem.at[slot]).wait()

            wg = w_v[g]
            for r in range(16):
                wr = wg[r]
                for k in range(FB // 16):
                    sb[r, pl.ds(k * 16, 16)] = gb[r, pl.ds(k * 16, 16)] * wr

            @pl.when(g + RING < NGRP)
            def _():
                gstart(g + RING, slot)

            dg = dst_v[g]
            pltpu.async_copy(sb, table.at[dg], ssem.at[slot], add=True)
            return 0
        lax.fori_loop(0, NGRP, mbody, 0)

        for slot in range(RING):
            pltpu.make_async_copy(sring.at[slot], table.at[pl.ds(0, 16)],
                                  ssem.at[slot]).wait()
        plsc.subcore_barrier()

        pltpu.sync_copy(table.at[pl.ds(s * BAND, BAND)],
                        out_hbm.at[pl.ds(base + s * BAND, BAND)])
        plsc.subcore_barrier()


@functools.lru_cache(maxsize=None)
def _get_sc_agg():
    return pl.kernel(
        _sc_agg_body,
        out_type=jax.ShapeDtypeStruct((NB4 * NP, FB), F32),
        mesh=_get_mesh(),
        compiler_params=pltpu.CompilerParams(needs_layout_passes=False,
                                             use_tc_tiling_on_sc=False),
        scratch_types=[
            pltpu.VMEM((NGRP, 16), I32),
            pltpu.VMEM((NGRP, 16), I32),
            pltpu.VMEM((NGRP, 16), F32),
            pltpu.VMEM((RING, 16, FB), F32),
            pltpu.VMEM((RING, 16, FB), F32),
            pltpu.SemaphoreType.DMA((RING,)),
            pltpu.SemaphoreType.DMA((RING,)),
            pltpu.VMEM_SHARED((NP, FB), F32),
        ],
    )


_BR = 512
_NBR = NP // _BR


def _lin_body(h_ref, w_ref, b_ref, as_ref, ad_ref, hw_ref, avs_ref, avd_ref):
    b = pl.program_id(1)
    hwb = jnp.dot(h_ref[...], w_ref[...], preferred_element_type=F32)
    hwb = hwb + b_ref[...]
    hw_ref[...] = hwb
    pa = jnp.dot(hwb, as_ref[...].reshape(FB), preferred_element_type=F32)
    pd = jnp.dot(hwb, ad_ref[...].reshape(FB), preferred_element_type=F32)

    @pl.when(b == 0)
    def _():
        avs_ref[...] = jnp.zeros((1, 1, _BR), F32)
        avd_ref[...] = jnp.zeros((1, 1, _BR), F32)
    avs_ref[...] += pa.reshape(1, 1, _BR)
    avd_ref[...] += pd.reshape(1, 1, _BR)


def _lin(h, W, bias, asv, adv):
    K = h.shape[1]
    hw, avs, avd = pl.pallas_call(
        _lin_body,
        grid=(_NBR, NB4),
        in_specs=[
            pl.BlockSpec((_BR, K), lambda i, b: (i, 0)),
            pl.BlockSpec((K, FB), lambda i, b: (0, b)),
            pl.BlockSpec((1, FB), lambda i, b: (0, b)),
            pl.BlockSpec((1, FB), lambda i, b: (0, b)),
            pl.BlockSpec((1, FB), lambda i, b: (0, b)),
        ],
        out_specs=[
            pl.BlockSpec((_BR, FB), lambda i, b: (b * _NBR + i, 0)),
            pl.BlockSpec((1, 1, _BR), lambda i, b: (i, 0, 0)),
            pl.BlockSpec((1, 1, _BR), lambda i, b: (i, 0, 0)),
        ],
        out_shape=[
            jax.ShapeDtypeStruct((NB4 * NP, FB), F32),
            jax.ShapeDtypeStruct((_NBR, 1, _BR), F32),
            jax.ShapeDtypeStruct((_NBR, 1, _BR), F32),
        ],
    )(h, W, bias.reshape(1, D_OUT), asv.reshape(1, D_OUT),
      adv.reshape(1, D_OUT))
    return hw, avs.reshape(NP), avd.reshape(NP)


def _act_body(o0, o1, o2, o3, den_ref, b_ref, h_ref):
    cat = jnp.concatenate([o0[...], o1[...], o2[...], o3[...]], axis=1)
    den = den_ref[...].reshape(_BR, 1)
    h_ref[...] = jnp.maximum(cat / (den + 1e-16) + b_ref[...], 0.0)


def _act(outflat, den, bias):
    specs = [pl.BlockSpec((_BR, FB), (lambda j: (lambda i: (j * _NBR + i, 0)))(j))
             for j in range(4)]
    return pl.pallas_call(
        _act_body,
        grid=(_NBR,),
        in_specs=specs + [
            pl.BlockSpec((1, 1, _BR), lambda i: (i, 0, 0)),
            pl.BlockSpec((1, D_OUT), lambda i: (0, 0)),
        ],
        out_specs=pl.BlockSpec((_BR, D_OUT), lambda i: (i, 0)),
        out_shape=jax.ShapeDtypeStruct((NP, D_OUT), F32),
    )(outflat, outflat, outflat, outflat, den.reshape(_NBR, 1, _BR),
      bias.reshape(1, D_OUT))


def _edge_scalar_body(ea_ref, u_ref, c_ref, s1_ref, s2_ref):
    sblk = jnp.dot(ea_ref[...], u_ref[...], preferred_element_type=F32)
    sblk = sblk + c_ref[...]
    s1_ref[...] = sblk[:, 0].reshape(1, 1, -1)
    s2_ref[...] = sblk[:, 1].reshape(1, 1, -1)


def _edge_scalars(edge_attr, U, c2):
    BE = 2000
    nb = E // BE
    D = edge_attr.shape[1]
    s1, s2 = pl.pallas_call(
        _edge_scalar_body,
        grid=(nb,),
        in_specs=[
            pl.BlockSpec((BE, D), lambda i: (i, 0)),
            pl.BlockSpec((D, 2), lambda i: (0, 0)),
            pl.BlockSpec((1, 2), lambda i: (0, 0)),
        ],
        out_specs=[
            pl.BlockSpec((1, 1, BE), lambda i: (i, 0, 0)),
            pl.BlockSpec((1, 1, BE), lambda i: (i, 0, 0)),
        ],
        out_shape=[
            jax.ShapeDtypeStruct((nb, 1, BE), F32),
            jax.ShapeDtypeStruct((nb, 1, BE), F32),
        ],
    )(edge_attr, U, c2.reshape(1, 2))
    return s1.reshape(E), s2.reshape(E)


def _pool_body(o0, o1, o2, o3, den_ref, b_ref, wp_ref, bp_ref, batch_ref,
               sums_ref):
    i = pl.program_id(0)
    cat = jnp.concatenate([o0[...], o1[...], o2[...], o3[...]], axis=1)
    den = den_ref[...].reshape(_BR, 1)
    hp = cat / (den + 1e-16) + b_ref[...]
    sc = jax.nn.sigmoid(jnp.dot(hp, wp_ref[...],
                                preferred_element_type=F32) + bp_ref[0, 0])
    xw = hp * sc
    bv = batch_ref[...].reshape(_BR, 1)
    oh = (bv == lax.broadcasted_iota(I32, (_BR, G), 1)).astype(F32)
    contrib = lax.dot_general(oh, xw, (((0,), (0,)), ((), ())),
                              preferred_element_type=F32)

    @pl.when(i == 0)
    def _():
        sums_ref[...] = jnp.zeros((G, D_OUT), F32)
    sums_ref[...] += contrib


def _pool(outflat, den, bias, wproj, bproj, batch3):
    specs = [pl.BlockSpec((_BR, FB), (lambda j: (lambda i: (j * _NBR + i, 0)))(j))
             for j in range(4)]
    return pl.pallas_call(
        _pool_body,
        grid=(_NBR,),
        in_specs=specs + [
            pl.BlockSpec((1, 1, _BR), lambda i: (i, 0, 0)),
            pl.BlockSpec((1, D_OUT), lambda i: (0, 0)),
            pl.BlockSpec((D_OUT, 1), lambda i: (0, 0)),
            pl.BlockSpec((1, 1), lambda i: (0, 0)),
            pl.BlockSpec((1, 1, _BR), lambda i: (i, 0, 0)),
        ],
        out_specs=pl.BlockSpec((G, D_OUT), lambda i: (0, 0)),
        out_shape=jax.ShapeDtypeStruct((G, D_OUT), F32),
    )(outflat, outflat, outflat, outflat, den.reshape(_NBR, 1, _BR),
      bias.reshape(1, D_OUT), wproj, bproj.reshape(1, 1), batch3)


def _fin_body(s1_ref, s2_ref, batch_ref, out_ref):
    bm = batch_ref[...].reshape(_NBR, _BR)
    acc = jnp.zeros((G,), F32)
    for r in range(_NBR):
        oh = (bm[r].reshape(_BR, 1)
              == lax.broadcasted_iota(I32, (_BR, G), 1)).astype(F32)
        acc = acc + jnp.sum(oh, axis=0)
    cnt = jnp.maximum(acc, 1.0).reshape(G, 1)
    out_ref[...] = jnp.concatenate(
        [s1_ref[...] / cnt, s2_ref[...] / cnt], axis=1)


def _fin(sums1, sums2, batch3):
    return pl.pallas_call(
        _fin_body,
        grid=(1,),
        in_specs=[
            pl.BlockSpec((G, D_OUT), lambda i: (0, 0)),
            pl.BlockSpec((G, D_OUT), lambda i: (0, 0)),
            pl.BlockSpec((_NBR, 1, _BR), lambda i: (0, 0, 0)),
        ],
        out_specs=pl.BlockSpec((G, 2 * D_OUT), lambda i: (0, 0)),
        out_shape=jax.ShapeDtypeStruct((G, 2 * D_OUT), F32),
    )(sums1, sums2, batch3)


def kernel(x, edge_index, edge_attr, batch, Wnp, bnp, Wep, bep,
           W1, as1, ad1, b1, We1, ae1, W2, as2, ad2, b2, We2, ae2,
           Wp1, asp1, adp1, bp1, wproj1, bproj1,
           Wp2, asp2, adp2, bp2, wproj2, bproj2):
    src = edge_index[0]
    dst = edge_index[1]

    v1 = We1 @ ae1
    v2 = We2 @ ae2
    U = jnp.stack([Wep @ v1, Wep @ v2], axis=1)
    c2 = jnp.stack([bep @ v1, bep @ v2])
    M1 = Wnp @ W1
    m1b = bnp @ W1

    loop_ids = jnp.arange(N, dtype=I32)
    pad_i = jnp.zeros((E2P - E2,), I32)
    srcg = jnp.concatenate([src, loop_ids, pad_i]).reshape(16, NGRP, 16)
    dstg = jnp.concatenate([dst, loop_ids, pad_i]).reshape(16, NGRP, 16)
    pad_f = jnp.full((E2P - E2,), -1e30, F32)
    dst_r = dst.reshape(16, CGRP, 16)
    zer = jnp.zeros((BAND, FB), F32)
    zeros512 = jnp.zeros((D_OUT,), F32)
    x_pad = jnp.concatenate([x, jnp.zeros((NP - N, x.shape[1]), F32)], axis=0)
    batch3 = jnp.concatenate(
        [batch, jnp.full((NP - N,), -1, I32)]).reshape(_NBR, 1, _BR)

    s1, s2 = _edge_scalars(edge_attr, U, c2)
    l1, l2 = _get_sc_loop_mean()(dst_r, s1.reshape(16, CGRP, 16),
                                 s2.reshape(16, CGRP, 16))
    se1 = jnp.concatenate([s1, l1[:N], pad_f]).reshape(16, NGRP, 16)
    se2 = jnp.concatenate([s2, l2[:N], pad_f]).reshape(16, NGRP, 16)
    se0 = jnp.concatenate([jnp.zeros((E2,), F32), pad_f]).reshape(16, NGRP, 16)

    hw1, avs1, avd1 = _lin(x_pad, M1, m1b, as1, ad1)
    w_out1, den1 = _get_sc_edge_w()(avs1, avd1, srcg, dstg, se1)
    out1 = _get_sc_agg()(hw1, srcg, dstg, w_out1, zer)
    h1 = _act(out1, den1, b1)

    hw2, avs2, avd2 = _lin(h1, W2, zeros512, as2, ad2)
    w_out2, den2 = _get_sc_edge_w()(avs2, avd2, srcg, dstg, se2)
    out2 = _get_sc_agg()(hw2, srcg, dstg, w_out2, zer)
    h2 = _act(out2, den2, b2)

    hwp1, avsp1, avdp1 = _lin(h2, Wp1, zeros512, asp1, adp1)
    hwp2, avsp2, avdp2 = _lin(h2, Wp2, zeros512, asp2, adp2)
    w_outp1, denp1 = _get_sc_edge_w()(avsp1, avdp1, srcg, dstg, se0)
    outp1 = _get_sc_agg()(hwp1, srcg, dstg, w_outp1, zer)
    sums1 = _pool(outp1, denp1, bp1, wproj1, bproj1, batch3)

    w_outp2, denp2 = _get_sc_edge_w()(avsp2, avdp2, srcg, dstg, se0)
    outp2 = _get_sc_agg()(hwp2, srcg, dstg, w_outp2, zer)
    sums2 = _pool(outp2, denp2, bp2, wproj2, bproj2, batch3)

    return _fin(sums1, sums2, batch3)


# speedup vs baseline: 1.0952x; 1.0859x over previous
import functools
import jax
import jax.numpy as jnp
from jax import lax
from jax.experimental import pallas as pl
from jax.experimental.pallas import tpu as pltpu
from jax.experimental.pallas import tpu_sc as plsc

F32 = jnp.float32
I32 = jnp.int32

N = 10000
E = 160000
G = 16
D_OUT = 512
NP = 10240
E2 = E + N
EPT = 10656
E2P = 16 * EPT
NGRP = EPT // 16
CPT = E // 16
CGRP = CPT // 16
BAND = NP // 16
NB4 = 4
FB = 128
RING = 4

@functools.lru_cache(maxsize=None)
def _get_mesh():
    return plsc.VectorSubcoreMesh(core_axis_name="c", subcore_axis_name="s")


def _sc_loop_mean_body(dst_hbm, s1_hbm, s2_hbm, l1_hbm, l2_hbm,
                       dst_v, s1_v, s2_v, cnt_v, su1_v, su2_v,
                       red_v, a_v, b_v, c_v, part):
    c = lax.axis_index("c")
    s = lax.axis_index("s")
    pltpu.sync_copy(dst_hbm.at[s], dst_v)
    pltpu.sync_copy(s1_hbm.at[s], s1_v)
    pltpu.sync_copy(s2_hbm.at[s], s2_v)

    zero16 = jnp.zeros((16,), F32)

    def zbody(i, _):
        cnt_v[pl.ds(i * 16, 16)] = zero16
        su1_v[pl.ds(i * 16, 16)] = zero16
        su2_v[pl.ds(i * 16, 16)] = zero16
        return 0
    lax.fori_loop(0, NP // 16, zbody, 0)

    one16 = jnp.full((16,), 1.0, F32)

    def ebody(g, _):
        dg = dst_v[g]
        plsc.addupdate_scatter(cnt_v, [dg], one16)
        plsc.addupdate_scatter(su1_v, [dg], s1_v[g])
        plsc.addupdate_scatter(su2_v, [dg], s2_v[g])
        return 0
    lax.fori_loop(0, CGRP, ebody, 0)

    pltpu.sync_copy(cnt_v, part.at[0, s])
    pltpu.sync_copy(su1_v, part.at[1, s])
    pltpu.sync_copy(su2_v, part.at[2, s])
    plsc.subcore_barrier()

    def _reduce(tab, outbuf):
        pltpu.sync_copy(part.at[tab, :, pl.ds(s * BAND, BAND)], red_v)

        def rbody(j, _):
            acc = jnp.zeros((16,), F32)
            for t in range(16):
                acc = acc + red_v[t, pl.ds(j * 16, 16)]
            outbuf[pl.ds(j * 16, 16)] = acc
            return 0
        lax.fori_loop(0, BAND // 16, rbody, 0)

    _reduce(0, c_v)
    _reduce(1, a_v)
    _reduce(2, b_v)

    def dbody(j, _):
        cc = jnp.maximum(c_v[pl.ds(j * 16, 16)], 1.0)
        a_v[pl.ds(j * 16, 16)] = a_v[pl.ds(j * 16, 16)] / cc
        b_v[pl.ds(j * 16, 16)] = b_v[pl.ds(j * 16, 16)] / cc
        return 0
    lax.fori_loop(0, BAND // 16, dbody, 0)

    @pl.when(c == 0)
    def _():
        pltpu.sync_copy(a_v, l1_hbm.at[pl.ds(s * BAND, BAND)])
        pltpu.sync_copy(b_v, l2_hbm.at[pl.ds(s * BAND, BAND)])


@functools.lru_cache(maxsize=None)
def _get_sc_loop_mean():
    return pl.kernel(
        _sc_loop_mean_body,
        out_type=(jax.ShapeDtypeStruct((NP,), F32),
                  jax.ShapeDtypeStruct((NP,), F32)),
        mesh=_get_mesh(),
        compiler_params=pltpu.CompilerParams(needs_layout_passes=False,
                                             use_tc_tiling_on_sc=False),
        scratch_types=[
        pltpu.VMEM((CGRP, 16), I32),
        pltpu.VMEM((CGRP, 16), F32),
        pltpu.VMEM((CGRP, 16), F32),
        pltpu.VMEM((NP,), F32),
        pltpu.VMEM((NP,), F32),
        pltpu.VMEM((NP,), F32),
        pltpu.VMEM((16, BAND), F32),
        pltpu.VMEM((BAND,), F32),
        pltpu.VMEM((BAND,), F32),
            pltpu.VMEM((BAND,), F32),
            pltpu.VMEM_SHARED((3, 16, NP), F32),
        ],
    )


def _sc_edge_w_body(avs_hbm, avd_hbm, src_hbm, dst_hbm, se_hbm,
                    w_hbm, den_hbm,
                    avs_v, avd_v, src_v, dst_v, se_v, w_v, den_v,
                    red_v, dout_v, denp):
    c = lax.axis_index("c")
    s = lax.axis_index("s")

    pltpu.sync_copy(avs_hbm, avs_v)
    pltpu.sync_copy(avd_hbm, avd_v)
    pltpu.sync_copy(src_hbm.at[s], src_v)
    pltpu.sync_copy(dst_hbm.at[s], dst_v)
    pltpu.sync_copy(se_hbm.at[s], se_v)

    zero16 = jnp.zeros((16,), F32)

    def zbody(i, _):
        den_v[pl.ds(i * 16, 16)] = zero16
        return 0
    lax.fori_loop(0, NP // 16, zbody, 0)

    def p1body(g, _):
        sg = src_v[g]
        dg = dst_v[g]
        a = (plsc.load_gather(avs_v, [sg]) + plsc.load_gather(avd_v, [dg])
             + se_v[g])
        a = jnp.where(a > 0, a, 0.2 * a)
        w = jnp.exp(a)
        w_v[g] = w
        plsc.addupdate_scatter(den_v, [dg], w)
        return 0
    lax.fori_loop(0, NGRP, p1body, 0)

    @pl.when(c == 0)
    def _():
        pltpu.sync_copy(w_v, w_hbm.at[s])

    pltpu.sync_copy(den_v, denp.at[s])
    plsc.subcore_barrier()

    pltpu.sync_copy(denp.at[:, pl.ds(s * BAND, BAND)], red_v)

    def rbody(j, _):
        acc = jnp.zeros((16,), F32)
        for t in range(16):
            acc = acc + red_v[t, pl.ds(j * 16, 16)]
        dout_v[pl.ds(j * 16, 16)] = acc
        return 0
    lax.fori_loop(0, BAND // 16, rbody, 0)

    @pl.when(c == 0)
    def _():
        pltpu.sync_copy(dout_v, den_hbm.at[pl.ds(s * BAND, BAND)])


@functools.lru_cache(maxsize=None)
def _get_sc_edge_w():
    return pl.kernel(
        _sc_edge_w_body,
        out_type=(jax.ShapeDtypeStruct((16, NGRP, 16), F32),
                  jax.ShapeDtypeStruct((NP,), F32)),
        mesh=_get_mesh(),
        compiler_params=pltpu.CompilerParams(needs_layout_passes=False,
                                             use_tc_tiling_on_sc=False),
        scratch_types=[
            pltpu.VMEM((NP,), F32),
            pltpu.VMEM((NP,), F32),
            pltpu.VMEM((NGRP, 16), I32),
            pltpu.VMEM((NGRP, 16), I32),
            pltpu.VMEM((NGRP, 16), F32),
            pltpu.VMEM((NGRP, 16), F32),
            pltpu.VMEM((NP,), F32),
            pltpu.VMEM((16, BAND), F32),
            pltpu.VMEM((BAND,), F32),
            pltpu.VMEM_SHARED((16, NP), F32),
        ],
    )


def _sc_agg_body(hw_hbm, src_hbm, dst_hbm, w_hbm, zer_hbm, out_hbm,
                 src_v, dst_v, w_v, gring, sring, gsem, ssem, table):
    c = lax.axis_index("c")
    s = lax.axis_index("s")

    pltpu.sync_copy(src_hbm.at[s], src_v)
    pltpu.sync_copy(dst_hbm.at[s], dst_v)
    pltpu.sync_copy(w_hbm.at[s], w_v)

    for bi in range(2):
        b = bi * 2 + c
        base = b * NP

        pltpu.sync_copy(zer_hbm, table.at[pl.ds(s * BAND, BAND)])
        plsc.subcore_barrier()

        def gstart(g, slot):
            idx = src_v[g] + base
            pltpu.async_copy(hw_hbm.at[idx], gring.at[slot], gsem.at[slot])

        for slot in range(RING):
            gstart(slot, slot)

        def mbody(g, _):
            slot = lax.rem(g, RING)
            gb = gring.at[slot]
            sb = sring.at[slot]
            pltpu.make_async_copy(hw_hbm.at[pl.ds(0, 16)], gb,
                                  gsem.at[slot]).wait()

            @pl.when(g >= RING)
            def _():
                pltpu.make_async_copy(sb, table.at[pl.ds(0, 16)],
                                      ssem.at[slot]).wait()

            wg = w_v[g]
            for r in range(16):
                wr = wg[r]
                for k in range(FB // 16):
                    sb[r, pl.ds(k * 16, 16)] = gb[r, pl.ds(k * 16, 16)] * wr

            @pl.when(g + RING < CGRP)
            def _():
                gstart(g + RING, slot)

            dg = dst_v[g]
            pltpu.async_copy(sb, table.at[dg], ssem.at[slot], add=True)
            return 0
        lax.fori_loop(0, CGRP, mbody, 0)

        for slot in range(RING):
            pltpu.make_async_copy(sring.at[slot], table.at[pl.ds(0, 16)],
                                  ssem.at[slot]).wait()
        plsc.subcore_barrier()

        pltpu.sync_copy(table.at[pl.ds(s * BAND, BAND)],
                        out_hbm.at[pl.ds(base + s * BAND, BAND)])
        plsc.subcore_barrier()


@functools.lru_cache(maxsize=None)
def _get_sc_agg():
    return pl.kernel(
        _sc_agg_body,
        out_type=jax.ShapeDtypeStruct((NB4 * NP, FB), F32),
        mesh=_get_mesh(),
        compiler_params=pltpu.CompilerParams(needs_layout_passes=False,
                                             use_tc_tiling_on_sc=False),
        scratch_types=[
            pltpu.VMEM((CGRP, 16), I32),
            pltpu.VMEM((CGRP, 16), I32),
            pltpu.VMEM((CGRP, 16), F32),
            pltpu.VMEM((RING, 16, FB), F32),
            pltpu.VMEM((RING, 16, FB), F32),
            pltpu.SemaphoreType.DMA((RING,)),
            pltpu.SemaphoreType.DMA((RING,)),
            pltpu.VMEM_SHARED((NP, FB), F32),
        ],
    )


_BR = 512
_NBR = NP // _BR


def _lin_body(h_ref, w_ref, b_ref, as_ref, ad_ref, hw_ref, avs_ref, avd_ref):
    b = pl.program_id(1)
    hwb = jnp.dot(h_ref[...], w_ref[...], preferred_element_type=F32)
    hwb = hwb + b_ref[...]
    hw_ref[...] = hwb
    pa = jnp.dot(hwb, as_ref[...].reshape(FB), preferred_element_type=F32)
    pd = jnp.dot(hwb, ad_ref[...].reshape(FB), preferred_element_type=F32)

    @pl.when(b == 0)
    def _():
        avs_ref[...] = jnp.zeros((1, 1, _BR), F32)
        avd_ref[...] = jnp.zeros((1, 1, _BR), F32)
    avs_ref[...] += pa.reshape(1, 1, _BR)
    avd_ref[...] += pd.reshape(1, 1, _BR)


def _lin(h, W, bias, asv, adv):
    K = h.shape[1]
    hw, avs, avd = pl.pallas_call(
        _lin_body,
        grid=(_NBR, NB4),
        in_specs=[
            pl.BlockSpec((_BR, K), lambda i, b: (i, 0)),
            pl.BlockSpec((K, FB), lambda i, b: (0, b)),
            pl.BlockSpec((1, FB), lambda i, b: (0, b)),
            pl.BlockSpec((1, FB), lambda i, b: (0, b)),
            pl.BlockSpec((1, FB), lambda i, b: (0, b)),
        ],
        out_specs=[
            pl.BlockSpec((_BR, FB), lambda i, b: (b * _NBR + i, 0)),
            pl.BlockSpec((1, 1, _BR), lambda i, b: (i, 0, 0)),
            pl.BlockSpec((1, 1, _BR), lambda i, b: (i, 0, 0)),
        ],
        out_shape=[
            jax.ShapeDtypeStruct((NB4 * NP, FB), F32),
            jax.ShapeDtypeStruct((_NBR, 1, _BR), F32),
            jax.ShapeDtypeStruct((_NBR, 1, _BR), F32),
        ],
    )(h, W, bias.reshape(1, D_OUT), asv.reshape(1, D_OUT),
      adv.reshape(1, D_OUT))
    return hw, avs.reshape(NP), avd.reshape(NP)


def _act_body(o0, o1, o2, o3, h0, h1, h2, h3, wl_ref, den_ref, b_ref,
              h_ref):
    cat = jnp.concatenate([o0[...], o1[...], o2[...], o3[...]], axis=1)
    hw = jnp.concatenate([h0[...], h1[...], h2[...], h3[...]], axis=1)
    wl = wl_ref[...].reshape(_BR, 1)
    den = den_ref[...].reshape(_BR, 1)
    h_ref[...] = jnp.maximum(
        (cat + wl * hw) / (den + 1e-16) + b_ref[...], 0.0)


def _act(outflat, hwflat, wl3, den, bias):
    specs = [pl.BlockSpec((_BR, FB), (lambda j: (lambda i: (j * _NBR + i, 0)))(j))
             for j in range(4)]
    return pl.pallas_call(
        _act_body,
        grid=(_NBR,),
        in_specs=specs + specs + [
            pl.BlockSpec((1, 1, _BR), lambda i: (i, 0, 0)),
            pl.BlockSpec((1, 1, _BR), lambda i: (i, 0, 0)),
            pl.BlockSpec((1, D_OUT), lambda i: (0, 0)),
        ],
        out_specs=pl.BlockSpec((_BR, D_OUT), lambda i: (i, 0)),
        out_shape=jax.ShapeDtypeStruct((NP, D_OUT), F32),
    )(outflat, outflat, outflat, outflat,
      hwflat, hwflat, hwflat, hwflat, wl3,
      den.reshape(_NBR, 1, _BR), bias.reshape(1, D_OUT))


def _edge_scalar_body(ea_ref, u_ref, c_ref, s1_ref, s2_ref):
    sblk = jnp.dot(ea_ref[...], u_ref[...], preferred_element_type=F32)
    sblk = sblk + c_ref[...]
    s1_ref[...] = sblk[:, 0].reshape(1, 1, -1)
    s2_ref[...] = sblk[:, 1].reshape(1, 1, -1)


def _edge_scalars(edge_attr, U, c2):
    BE = 2000
    nb = E // BE
    D = edge_attr.shape[1]
    s1, s2 = pl.pallas_call(
        _edge_scalar_body,
        grid=(nb,),
        in_specs=[
            pl.BlockSpec((BE, D), lambda i: (i, 0)),
            pl.BlockSpec((D, 2), lambda i: (0, 0)),
            pl.BlockSpec((1, 2), lambda i: (0, 0)),
        ],
        out_specs=[
            pl.BlockSpec((1, 1, BE), lambda i: (i, 0, 0)),
            pl.BlockSpec((1, 1, BE), lambda i: (i, 0, 0)),
        ],
        out_shape=[
            jax.ShapeDtypeStruct((nb, 1, BE), F32),
            jax.ShapeDtypeStruct((nb, 1, BE), F32),
        ],
    )(edge_attr, U, c2.reshape(1, 2))
    return s1.reshape(E), s2.reshape(E)


def _pool_body(o0, o1, o2, o3, h0, h1, h2, h3, wl_ref, den_ref, b_ref,
               wp_ref, bp_ref, batch_ref, sums_ref):
    i = pl.program_id(0)
    cat = jnp.concatenate([o0[...], o1[...], o2[...], o3[...]], axis=1)
    hw = jnp.concatenate([h0[...], h1[...], h2[...], h3[...]], axis=1)
    wl = wl_ref[...].reshape(_BR, 1)
    den = den_ref[...].reshape(_BR, 1)
    hp = (cat + wl * hw) / (den + 1e-16) + b_ref[...]
    sc = jax.nn.sigmoid(jnp.dot(hp, wp_ref[...],
                                preferred_element_type=F32) + bp_ref[0, 0])
    xw = hp * sc
    bv = batch_ref[...].reshape(_BR, 1)
    oh = (bv == lax.broadcasted_iota(I32, (_BR, G), 1)).astype(F32)
    contrib = lax.dot_general(oh, xw, (((0,), (0,)), ((), ())),
                              preferred_element_type=F32)

    @pl.when(i == 0)
    def _():
        sums_ref[...] = jnp.zeros((G, D_OUT), F32)
    sums_ref[...] += contrib


def _pool(outflat, hwflat, wl3, den, bias, wproj, bproj, batch3):
    specs = [pl.BlockSpec((_BR, FB), (lambda j: (lambda i: (j * _NBR + i, 0)))(j))
             for j in range(4)]
    return pl.pallas_call(
        _pool_body,
        grid=(_NBR,),
        in_specs=specs + specs + [
            pl.BlockSpec((1, 1, _BR), lambda i: (i, 0, 0)),
            pl.BlockSpec((1, 1, _BR), lambda i: (i, 0, 0)),
            pl.BlockSpec((1, D_OUT), lambda i: (0, 0)),
            pl.BlockSpec((D_OUT, 1), lambda i: (0, 0)),
            pl.BlockSpec((1, 1), lambda i: (0, 0)),
            pl.BlockSpec((1, 1, _BR), lambda i: (i, 0, 0)),
        ],
        out_specs=pl.BlockSpec((G, D_OUT), lambda i: (0, 0)),
        out_shape=jax.ShapeDtypeStruct((G, D_OUT), F32),
    )(outflat, outflat, outflat, outflat,
      hwflat, hwflat, hwflat, hwflat, wl3,
      den.reshape(_NBR, 1, _BR),
      bias.reshape(1, D_OUT), wproj, bproj.reshape(1, 1), batch3)


def _fin_body(s1_ref, s2_ref, batch_ref, out_ref):
    bm = batch_ref[...].reshape(_NBR, _BR)
    acc = jnp.zeros((G,), F32)
    for r in range(_NBR):
        oh = (bm[r].reshape(_BR, 1)
              == lax.broadcasted_iota(I32, (_BR, G), 1)).astype(F32)
        acc = acc + jnp.sum(oh, axis=0)
    cnt = jnp.maximum(acc, 1.0).reshape(G, 1)
    out_ref[...] = jnp.concatenate(
        [s1_ref[...] / cnt, s2_ref[...] / cnt], axis=1)


def _fin(sums1, sums2, batch3):
    return pl.pallas_call(
        _fin_body,
        grid=(1,),
        in_specs=[
            pl.BlockSpec((G, D_OUT), lambda i: (0, 0)),
            pl.BlockSpec((G, D_OUT), lambda i: (0, 0)),
            pl.BlockSpec((_NBR, 1, _BR), lambda i: (0, 0, 0)),
        ],
        out_specs=pl.BlockSpec((G, 2 * D_OUT), lambda i: (0, 0)),
        out_shape=jax.ShapeDtypeStruct((G, 2 * D_OUT), F32),
    )(sums1, sums2, batch3)


def kernel(x, edge_index, edge_attr, batch, Wnp, bnp, Wep, bep,
           W1, as1, ad1, b1, We1, ae1, W2, as2, ad2, b2, We2, ae2,
           Wp1, asp1, adp1, bp1, wproj1, bproj1,
           Wp2, asp2, adp2, bp2, wproj2, bproj2):
    src = edge_index[0]
    dst = edge_index[1]

    v1 = We1 @ ae1
    v2 = We2 @ ae2
    U = jnp.stack([Wep @ v1, Wep @ v2], axis=1)
    c2 = jnp.stack([bep @ v1, bep @ v2])
    M1 = Wnp @ W1
    m1b = bnp @ W1

    loop_ids = jnp.arange(N, dtype=I32)
    pad_i = jnp.zeros((E2P - E2,), I32)
    srcg = jnp.concatenate([src, loop_ids, pad_i]).reshape(16, NGRP, 16)
    dstg = jnp.concatenate([dst, loop_ids, pad_i]).reshape(16, NGRP, 16)
    pad_f = jnp.full((E2P - E2,), -1e30, F32)
    dst_r = dst.reshape(16, CGRP, 16)
    zer = jnp.zeros((BAND, FB), F32)
    zeros512 = jnp.zeros((D_OUT,), F32)
    x_pad = jnp.concatenate([x, jnp.zeros((NP - N, x.shape[1]), F32)], axis=0)
    batch3 = jnp.concatenate(
        [batch, jnp.full((NP - N,), -1, I32)]).reshape(_NBR, 1, _BR)

    s1, s2 = _edge_scalars(edge_attr, U, c2)
    l1, l2 = _get_sc_loop_mean()(dst_r, s1.reshape(16, CGRP, 16),
                                 s2.reshape(16, CGRP, 16))
    se1 = jnp.concatenate([s1, l1[:N], pad_f]).reshape(16, NGRP, 16)
    se2 = jnp.concatenate([s2, l2[:N], pad_f]).reshape(16, NGRP, 16)
    se0 = jnp.concatenate([jnp.zeros((E2,), F32), pad_f]).reshape(16, NGRP, 16)

    srcga = src.reshape(16, CGRP, 16)
    dstga = dst.reshape(16, CGRP, 16)

    def _wsplit(wfull):
        wflat = wfull.reshape(E2P)
        wa = wflat[:E].reshape(16, CGRP, 16)
        wl3 = jnp.concatenate([wflat[E:E2], jnp.zeros((NP - N,), F32)])
        return wa, wl3.reshape(_NBR, 1, _BR)

    hw1, avs1, avd1 = _lin(x_pad, M1, m1b, as1, ad1)
    w_out1, den1 = _get_sc_edge_w()(avs1, avd1, srcg, dstg, se1)
    wa1, wl1 = _wsplit(w_out1)
    out1 = _get_sc_agg()(hw1, srcga, dstga, wa1, zer)
    h1 = _act(out1, hw1, wl1, den1, b1)

    hw2, avs2, avd2 = _lin(h1, W2, zeros512, as2, ad2)
    w_out2, den2 = _get_sc_edge_w()(avs2, avd2, srcg, dstg, se2)
    wa2, wl2 = _wsplit(w_out2)
    out2 = _get_sc_agg()(hw2, srcga, dstga, wa2, zer)
    h2 = _act(out2, hw2, wl2, den2, b2)

    hwp1, avsp1, avdp1 = _lin(h2, Wp1, zeros512, asp1, adp1)
    hwp2, avsp2, avdp2 = _lin(h2, Wp2, zeros512, asp2, adp2)
    w_outp1, denp1 = _get_sc_edge_w()(avsp1, avdp1, srcg, dstg, se0)
    wap1, wlp1 = _wsplit(w_outp1)
    outp1 = _get_sc_agg()(hwp1, srcga, dstga, wap1, zer)
    sums1 = _pool(outp1, hwp1, wlp1, denp1, bp1, wproj1, bproj1, batch3)

    w_outp2, denp2 = _get_sc_edge_w()(avsp2, avdp2, srcg, dstg, se0)
    wap2, wlp2 = _wsplit(w_outp2)
    outp2 = _get_sc_agg()(hwp2, srcga, dstga, wap2, zer)
    sums2 = _pool(outp2, hwp2, wlp2, denp2, bp2, wproj2, bproj2, batch3)

    return _fin(sums1, sums2, batch3)


# speedup vs baseline: 1.0973x; 1.0019x over previous
import functools
import jax
import jax.numpy as jnp
from jax import lax
from jax.experimental import pallas as pl
from jax.experimental.pallas import tpu as pltpu
from jax.experimental.pallas import tpu_sc as plsc

F32 = jnp.float32
I32 = jnp.int32

N = 10000
E = 160000
G = 16
D_OUT = 512
NP = 10240
E2 = E + N
EPT = 10656
E2P = 16 * EPT
NGRP = EPT // 16
CPT = E // 16
CGRP = CPT // 16
BAND = NP // 16
NB4 = 4
FB = 128
RING = 4

@functools.lru_cache(maxsize=None)
def _get_mesh():
    return plsc.VectorSubcoreMesh(core_axis_name="c", subcore_axis_name="s")


def _sc_loop_mean_body(dst_hbm, s1_hbm, s2_hbm, l1_hbm, l2_hbm,
                       dst_v, s1_v, s2_v, cnt_v, su1_v, su2_v,
                       red_v, a_v, b_v, c_v, part):
    c = lax.axis_index("c")
    s = lax.axis_index("s")
    pltpu.sync_copy(dst_hbm.at[s], dst_v)
    pltpu.sync_copy(s1_hbm.at[s], s1_v)
    pltpu.sync_copy(s2_hbm.at[s], s2_v)

    zero16 = jnp.zeros((16,), F32)

    def zbody(i, _):
        cnt_v[pl.ds(i * 16, 16)] = zero16
        su1_v[pl.ds(i * 16, 16)] = zero16
        su2_v[pl.ds(i * 16, 16)] = zero16
        return 0
    lax.fori_loop(0, NP // 16, zbody, 0)

    one16 = jnp.full((16,), 1.0, F32)

    def ebody(g, _):
        dg = dst_v[g]
        plsc.addupdate_scatter(cnt_v, [dg], one16)
        plsc.addupdate_scatter(su1_v, [dg], s1_v[g])
        plsc.addupdate_scatter(su2_v, [dg], s2_v[g])
        return 0
    lax.fori_loop(0, CGRP, ebody, 0)

    pltpu.sync_copy(cnt_v, part.at[0, s])
    pltpu.sync_copy(su1_v, part.at[1, s])
    pltpu.sync_copy(su2_v, part.at[2, s])
    plsc.subcore_barrier()

    def _reduce(tab, outbuf):
        pltpu.sync_copy(part.at[tab, :, pl.ds(s * BAND, BAND)], red_v)

        def rbody(j, _):
            acc = jnp.zeros((16,), F32)
            for t in range(16):
                acc = acc + red_v[t, pl.ds(j * 16, 16)]
            outbuf[pl.ds(j * 16, 16)] = acc
            return 0
        lax.fori_loop(0, BAND // 16, rbody, 0)

    _reduce(0, c_v)
    _reduce(1, a_v)
    _reduce(2, b_v)

    def dbody(j, _):
        cc = jnp.maximum(c_v[pl.ds(j * 16, 16)], 1.0)
        a_v[pl.ds(j * 16, 16)] = a_v[pl.ds(j * 16, 16)] / cc
        b_v[pl.ds(j * 16, 16)] = b_v[pl.ds(j * 16, 16)] / cc
        return 0
    lax.fori_loop(0, BAND // 16, dbody, 0)

    @pl.when(c == 0)
    def _():
        pltpu.sync_copy(a_v, l1_hbm.at[pl.ds(s * BAND, BAND)])
        pltpu.sync_copy(b_v, l2_hbm.at[pl.ds(s * BAND, BAND)])


@functools.lru_cache(maxsize=None)
def _get_sc_loop_mean():
    return pl.kernel(
        _sc_loop_mean_body,
        out_type=(jax.ShapeDtypeStruct((NP,), F32),
                  jax.ShapeDtypeStruct((NP,), F32)),
        mesh=_get_mesh(),
        compiler_params=pltpu.CompilerParams(needs_layout_passes=False,
                                             use_tc_tiling_on_sc=False),
        scratch_types=[
        pltpu.VMEM((CGRP, 16), I32),
        pltpu.VMEM((CGRP, 16), F32),
        pltpu.VMEM((CGRP, 16), F32),
        pltpu.VMEM((NP,), F32),
        pltpu.VMEM((NP,), F32),
        pltpu.VMEM((NP,), F32),
        pltpu.VMEM((16, BAND), F32),
        pltpu.VMEM((BAND,), F32),
        pltpu.VMEM((BAND,), F32),
            pltpu.VMEM((BAND,), F32),
            pltpu.VMEM_SHARED((3, 16, NP), F32),
        ],
    )


def _sc_edge_w_body(avs_hbm, avd_hbm, src_hbm, dst_hbm, se_hbm,
                    w_hbm, den_hbm,
                    avs_v, avd_v, src_v, dst_v, se_v, w_v, den_v,
                    red_v, dout_v, denp):
    c = lax.axis_index("c")
    s = lax.axis_index("s")

    pltpu.sync_copy(avs_hbm, avs_v)
    pltpu.sync_copy(avd_hbm, avd_v)
    pltpu.sync_copy(src_hbm.at[s], src_v)
    pltpu.sync_copy(dst_hbm.at[s], dst_v)
    pltpu.sync_copy(se_hbm.at[s], se_v)

    zero16 = jnp.zeros((16,), F32)

    def zbody(i, _):
        den_v[pl.ds(i * 16, 16)] = zero16
        return 0
    lax.fori_loop(0, NP // 16, zbody, 0)

    def p1body(g, _):
        sg = src_v[g]
        dg = dst_v[g]
        a = (plsc.load_gather(avs_v, [sg]) + plsc.load_gather(avd_v, [dg])
             + se_v[g])
        a = jnp.where(a > 0, a, 0.2 * a)
        w = jnp.exp(a)
        w_v[g] = w
        plsc.addupdate_scatter(den_v, [dg], w)
        return 0
    lax.fori_loop(0, NGRP, p1body, 0)

    @pl.when(c == 0)
    def _():
        pltpu.sync_copy(w_v, w_hbm.at[s])

    pltpu.sync_copy(den_v, denp.at[s])
    plsc.subcore_barrier()

    pltpu.sync_copy(denp.at[:, pl.ds(s * BAND, BAND)], red_v)

    def rbody(j, _):
        acc = jnp.zeros((16,), F32)
        for t in range(16):
            acc = acc + red_v[t, pl.ds(j * 16, 16)]
        dout_v[pl.ds(j * 16, 16)] = acc
        return 0
    lax.fori_loop(0, BAND // 16, rbody, 0)

    @pl.when(c == 0)
    def _():
        pltpu.sync_copy(dout_v, den_hbm.at[pl.ds(s * BAND, BAND)])


@functools.lru_cache(maxsize=None)
def _get_sc_edge_w():
    return pl.kernel(
        _sc_edge_w_body,
        out_type=(jax.ShapeDtypeStruct((16, NGRP, 16), F32),
                  jax.ShapeDtypeStruct((NP,), F32)),
        mesh=_get_mesh(),
        compiler_params=pltpu.CompilerParams(needs_layout_passes=False,
                                             use_tc_tiling_on_sc=False),
        scratch_types=[
            pltpu.VMEM((NP,), F32),
            pltpu.VMEM((NP,), F32),
            pltpu.VMEM((NGRP, 16), I32),
            pltpu.VMEM((NGRP, 16), I32),
            pltpu.VMEM((NGRP, 16), F32),
            pltpu.VMEM((NGRP, 16), F32),
            pltpu.VMEM((NP,), F32),
            pltpu.VMEM((16, BAND), F32),
            pltpu.VMEM((BAND,), F32),
            pltpu.VMEM_SHARED((16, NP), F32),
        ],
    )


def _sc_agg_body(hw_hbm, src_hbm, dst_hbm, w_hbm, zer_hbm, out_hbm,
                 src_v, dst_v, w_v, gring, sring, gsem, ssem, table):
    c = lax.axis_index("c")
    s = lax.axis_index("s")

    pltpu.sync_copy(src_hbm.at[s], src_v)
    pltpu.sync_copy(dst_hbm.at[s], dst_v)
    pltpu.sync_copy(w_hbm.at[s], w_v)

    for bi in range(2):
        b = bi * 2 + c
        base = b * NP

        pltpu.sync_copy(zer_hbm, table.at[pl.ds(s * BAND, BAND)])
        plsc.subcore_barrier()

        def gstart(g, slot):
            idx = src_v[g] + base
            pltpu.async_copy(hw_hbm.at[idx], gring.at[slot], gsem.at[slot])

        for slot in range(RING):
            gstart(slot, slot)

        def mbody(g, _):
            slot = lax.rem(g, RING)
            gb = gring.at[slot]
            sb = sring.at[slot]
            pltpu.make_async_copy(hw_hbm.at[pl.ds(0, 16)], gb,
                                  gsem.at[slot]).wait()

            @pl.when(g >= RING)
            def _():
                pltpu.make_async_copy(sb, table.at[pl.ds(0, 16)],
                                      ssem.at[slot]).wait()

            wg = w_v[g]
            for r in range(16):
                wr = wg[r]
                for k in range(FB // 16):
                    sb[r, pl.ds(k * 16, 16)] = gb[r, pl.ds(k * 16, 16)] * wr

            @pl.when(g + RING < CGRP)
            def _():
                gstart(g + RING, slot)

            dg = dst_v[g]
            pltpu.async_copy(sb, table.at[dg], ssem.at[slot], add=True)
            return 0
        lax.fori_loop(0, CGRP, mbody, 0)

        for slot in range(RING):
            pltpu.make_async_copy(sring.at[slot], table.at[pl.ds(0, 16)],
                                  ssem.at[slot]).wait()
        plsc.subcore_barrier()

        pltpu.sync_copy(table.at[pl.ds(s * BAND, BAND)],
                        out_hbm.at[pl.ds(base + s * BAND, BAND)])
        plsc.subcore_barrier()


@functools.lru_cache(maxsize=None)
def _get_sc_agg():
    return pl.kernel(
        _sc_agg_body,
        out_type=jax.ShapeDtypeStruct((NB4 * NP, FB), F32),
        mesh=_get_mesh(),
        compiler_params=pltpu.CompilerParams(needs_layout_passes=False,
                                             use_tc_tiling_on_sc=False),
        scratch_types=[
            pltpu.VMEM((CGRP, 16), I32),
            pltpu.VMEM((CGRP, 16), I32),
            pltpu.VMEM((CGRP, 16), F32),
            pltpu.VMEM((RING, 16, FB), F32),
            pltpu.VMEM((RING, 16, FB), F32),
            pltpu.SemaphoreType.DMA((RING,)),
            pltpu.SemaphoreType.DMA((RING,)),
            pltpu.VMEM_SHARED((NP, FB), F32),
        ],
    )


_BR = 512
_NBR = NP // _BR


def _lin_body(h_ref, w_ref, b_ref, as_ref, ad_ref, hw_ref, avs_ref, avd_ref):
    b = pl.program_id(1)
    hwb = jnp.dot(h_ref[...], w_ref[...], preferred_element_type=F32)
    hwb = hwb + b_ref[...]
    hw_ref[...] = hwb
    pa = jnp.dot(hwb, as_ref[...].reshape(FB), preferred_element_type=F32)
    pd = jnp.dot(hwb, ad_ref[...].reshape(FB), preferred_element_type=F32)

    @pl.when(b == 0)
    def _():
        avs_ref[...] = jnp.zeros((1, 1, _BR), F32)
        avd_ref[...] = jnp.zeros((1, 1, _BR), F32)
    avs_ref[...] += pa.reshape(1, 1, _BR)
    avd_ref[...] += pd.reshape(1, 1, _BR)


def _lin(h, W, bias, asv, adv):
    K = h.shape[1]
    hw, avs, avd = pl.pallas_call(
        _lin_body,
        grid=(_NBR, NB4),
        in_specs=[
            pl.BlockSpec((_BR, K), lambda i, b: (i, 0)),
            pl.BlockSpec((K, FB), lambda i, b: (0, b)),
            pl.BlockSpec((1, FB), lambda i, b: (0, b)),
            pl.BlockSpec((1, FB), lambda i, b: (0, b)),
            pl.BlockSpec((1, FB), lambda i, b: (0, b)),
        ],
        out_specs=[
            pl.BlockSpec((_BR, FB), lambda i, b: (b * _NBR + i, 0)),
            pl.BlockSpec((1, 1, _BR), lambda i, b: (i, 0, 0)),
            pl.BlockSpec((1, 1, _BR), lambda i, b: (i, 0, 0)),
        ],
        out_shape=[
            jax.ShapeDtypeStruct((NB4 * NP, FB), F32),
            jax.ShapeDtypeStruct((_NBR, 1, _BR), F32),
            jax.ShapeDtypeStruct((_NBR, 1, _BR), F32),
        ],
    )(h, W, bias.reshape(1, D_OUT), asv.reshape(1, D_OUT),
      adv.reshape(1, D_OUT))
    return hw, avs.reshape(NP), avd.reshape(NP)


def _lin_act_body(o0, o1, o2, o3, h0, h1, h2, h3, wl_ref, den_ref,
                  bp_ref, w_ref, as_ref, ad_ref, hw_ref, avs_ref, avd_ref):
    b = pl.program_id(1)
    cat = jnp.concatenate([o0[...], o1[...], o2[...], o3[...]], axis=1)
    hwp = jnp.concatenate([h0[...], h1[...], h2[...], h3[...]], axis=1)
    wl = wl_ref[...].reshape(_BR, 1)
    den = den_ref[...].reshape(_BR, 1)
    hin = jnp.maximum((cat + wl * hwp) / (den + 1e-16) + bp_ref[...], 0.0)
    hwb = jnp.dot(hin, w_ref[...], preferred_element_type=F32)
    hw_ref[...] = hwb
    pa = jnp.dot(hwb, as_ref[...].reshape(FB), preferred_element_type=F32)
    pd = jnp.dot(hwb, ad_ref[...].reshape(FB), preferred_element_type=F32)

    @pl.when(b == 0)
    def _():
        avs_ref[...] = jnp.zeros((1, 1, _BR), F32)
        avd_ref[...] = jnp.zeros((1, 1, _BR), F32)
    avs_ref[...] += pa.reshape(1, 1, _BR)
    avd_ref[...] += pd.reshape(1, 1, _BR)


def _lin_act(outflat, hwflat, wl3, den, bprev, W, asv, adv):
    specs = [pl.BlockSpec((_BR, FB),
                          (lambda j: (lambda i, b: (j * _NBR + i, 0)))(j))
             for j in range(4)]
    hw, avs, avd = pl.pallas_call(
        _lin_act_body,
        grid=(_NBR, NB4),
        in_specs=specs + specs + [
            pl.BlockSpec((1, 1, _BR), lambda i, b: (i, 0, 0)),
            pl.BlockSpec((1, 1, _BR), lambda i, b: (i, 0, 0)),
            pl.BlockSpec((1, D_OUT), lambda i, b: (0, 0)),
            pl.BlockSpec((D_OUT, FB), lambda i, b: (0, b)),
            pl.BlockSpec((1, FB), lambda i, b: (0, b)),
            pl.BlockSpec((1, FB), lambda i, b: (0, b)),
        ],
        out_specs=[
            pl.BlockSpec((_BR, FB), lambda i, b: (b * _NBR + i, 0)),
            pl.BlockSpec((1, 1, _BR), lambda i, b: (i, 0, 0)),
            pl.BlockSpec((1, 1, _BR), lambda i, b: (i, 0, 0)),
        ],
        out_shape=[
            jax.ShapeDtypeStruct((NB4 * NP, FB), F32),
            jax.ShapeDtypeStruct((_NBR, 1, _BR), F32),
            jax.ShapeDtypeStruct((_NBR, 1, _BR), F32),
        ],
    )(outflat, outflat, outflat, outflat,
      hwflat, hwflat, hwflat, hwflat, wl3, den.reshape(_NBR, 1, _BR),
      bprev.reshape(1, D_OUT), W, asv.reshape(1, D_OUT),
      adv.reshape(1, D_OUT))
    return hw, avs.reshape(NP), avd.reshape(NP)


def _act_body(o0, o1, o2, o3, h0, h1, h2, h3, wl_ref, den_ref, b_ref,
              h_ref):
    cat = jnp.concatenate([o0[...], o1[...], o2[...], o3[...]], axis=1)
    hw = jnp.concatenate([h0[...], h1[...], h2[...], h3[...]], axis=1)
    wl = wl_ref[...].reshape(_BR, 1)
    den = den_ref[...].reshape(_BR, 1)
    h_ref[...] = jnp.maximum(
        (cat + wl * hw) / (den + 1e-16) + b_ref[...], 0.0)


def _act(outflat, hwflat, wl3, den, bias):
    specs = [pl.BlockSpec((_BR, FB), (lambda j: (lambda i: (j * _NBR + i, 0)))(j))
             for j in range(4)]
    return pl.pallas_call(
        _act_body,
        grid=(_NBR,),
        in_specs=specs + specs + [
            pl.BlockSpec((1, 1, _BR), lambda i: (i, 0, 0)),
            pl.BlockSpec((1, 1, _BR), lambda i: (i, 0, 0)),
            pl.BlockSpec((1, D_OUT), lambda i: (0, 0)),
        ],
        out_specs=pl.BlockSpec((_BR, D_OUT), lambda i: (i, 0)),
        out_shape=jax.ShapeDtypeStruct((NP, D_OUT), F32),
    )(outflat, outflat, outflat, outflat,
      hwflat, hwflat, hwflat, hwflat, wl3,
      den.reshape(_NBR, 1, _BR), bias.reshape(1, D_OUT))


def _edge_scalar_body(ea_ref, u_ref, c_ref, s1_ref, s2_ref):
    sblk = jnp.dot(ea_ref[...], u_ref[...], preferred_element_type=F32)
    sblk = sblk + c_ref[...]
    s1_ref[...] = sblk[:, 0].reshape(1, 1, -1)
    s2_ref[...] = sblk[:, 1].reshape(1, 1, -1)


def _edge_scalars(edge_attr, U, c2):
    BE = 2000
    nb = E // BE
    D = edge_attr.shape[1]
    s1, s2 = pl.pallas_call(
        _edge_scalar_body,
        grid=(nb,),
        in_specs=[
            pl.BlockSpec((BE, D), lambda i: (i, 0)),
            pl.BlockSpec((D, 2), lambda i: (0, 0)),
            pl.BlockSpec((1, 2), lambda i: (0, 0)),
        ],
        out_specs=[
            pl.BlockSpec((1, 1, BE), lambda i: (i, 0, 0)),
            pl.BlockSpec((1, 1, BE), lambda i: (i, 0, 0)),
        ],
        out_shape=[
            jax.ShapeDtypeStruct((nb, 1, BE), F32),
            jax.ShapeDtypeStruct((nb, 1, BE), F32),
        ],
    )(edge_attr, U, c2.reshape(1, 2))
    return s1.reshape(E), s2.reshape(E)


def _pool_body(o0, o1, o2, o3, h0, h1, h2, h3, wl_ref, den_ref, b_ref,
               wp_ref, bp_ref, batch_ref, sums_ref):
    i = pl.program_id(0)
    cat = jnp.concatenate([o0[...], o1[...], o2[...], o3[...]], axis=1)
    hw = jnp.concatenate([h0[...], h1[...], h2[...], h3[...]], axis=1)
    wl = wl_ref[...].reshape(_BR, 1)
    den = den_ref[...].reshape(_BR, 1)
    hp = (cat + wl * hw) / (den + 1e-16) + b_ref[...]
    sc = jax.nn.sigmoid(jnp.dot(hp, wp_ref[...],
                                preferred_element_type=F32) + bp_ref[0, 0])
    xw = hp * sc
    bv = batch_ref[...].reshape(_BR, 1)
    oh = (bv == lax.broadcasted_iota(I32, (_BR, G), 1)).astype(F32)
    contrib = lax.dot_general(oh, xw, (((0,), (0,)), ((), ())),
                              preferred_element_type=F32)

    @pl.when(i == 0)
    def _():
        sums_ref[...] = jnp.zeros((G, D_OUT), F32)
    sums_ref[...] += contrib


def _pool(outflat, hwflat, wl3, den, bias, wproj, bproj, batch3):
    specs = [pl.BlockSpec((_BR, FB), (lambda j: (lambda i: (j * _NBR + i, 0)))(j))
             for j in range(4)]
    return pl.pallas_call(
        _pool_body,
        grid=(_NBR,),
        in_specs=specs + specs + [
            pl.BlockSpec((1, 1, _BR), lambda i: (i, 0, 0)),
            pl.BlockSpec((1, 1, _BR), lambda i: (i, 0, 0)),
            pl.BlockSpec((1, D_OUT), lambda i: (0, 0)),
            pl.BlockSpec((D_OUT, 1), lambda i: (0, 0)),
            pl.BlockSpec((1, 1), lambda i: (0, 0)),
            pl.BlockSpec((1, 1, _BR), lambda i: (i, 0, 0)),
        ],
        out_specs=pl.BlockSpec((G, D_OUT), lambda i: (0, 0)),
        out_shape=jax.ShapeDtypeStruct((G, D_OUT), F32),
    )(outflat, outflat, outflat, outflat,
      hwflat, hwflat, hwflat, hwflat, wl3,
      den.reshape(_NBR, 1, _BR),
      bias.reshape(1, D_OUT), wproj, bproj.reshape(1, 1), batch3)


def _fin_body(s1_ref, s2_ref, batch_ref, out_ref):
    bm = batch_ref[...].reshape(_NBR, _BR)
    acc = jnp.zeros((G,), F32)
    for r in range(_NBR):
        oh = (bm[r].reshape(_BR, 1)
              == lax.broadcasted_iota(I32, (_BR, G), 1)).astype(F32)
        acc = acc + jnp.sum(oh, axis=0)
    cnt = jnp.maximum(acc, 1.0).reshape(G, 1)
    out_ref[...] = jnp.concatenate(
        [s1_ref[...] / cnt, s2_ref[...] / cnt], axis=1)


def _fin(sums1, sums2, batch3):
    return pl.pallas_call(
        _fin_body,
        grid=(1,),
        in_specs=[
            pl.BlockSpec((G, D_OUT), lambda i: (0, 0)),
            pl.BlockSpec((G, D_OUT), lambda i: (0, 0)),
            pl.BlockSpec((_NBR, 1, _BR), lambda i: (0, 0, 0)),
        ],
        out_specs=pl.BlockSpec((G, 2 * D_OUT), lambda i: (0, 0)),
        out_shape=jax.ShapeDtypeStruct((G, 2 * D_OUT), F32),
    )(sums1, sums2, batch3)


def kernel(x, edge_index, edge_attr, batch, Wnp, bnp, Wep, bep,
           W1, as1, ad1, b1, We1, ae1, W2, as2, ad2, b2, We2, ae2,
           Wp1, asp1, adp1, bp1, wproj1, bproj1,
           Wp2, asp2, adp2, bp2, wproj2, bproj2):
    src = edge_index[0]
    dst = edge_index[1]

    v1 = We1 @ ae1
    v2 = We2 @ ae2
    U = jnp.stack([Wep @ v1, Wep @ v2], axis=1)
    c2 = jnp.stack([bep @ v1, bep @ v2])
    M1 = Wnp @ W1
    m1b = bnp @ W1

    loop_ids = jnp.arange(N, dtype=I32)
    pad_i = jnp.zeros((E2P - E2,), I32)
    srcg = jnp.concatenate([src, loop_ids, pad_i]).reshape(16, NGRP, 16)
    dstg = jnp.concatenate([dst, loop_ids, pad_i]).reshape(16, NGRP, 16)
    pad_f = jnp.full((E2P - E2,), -1e30, F32)
    dst_r = dst.reshape(16, CGRP, 16)
    zer = jnp.zeros((BAND, FB), F32)
    zeros512 = jnp.zeros((D_OUT,), F32)
    x_pad = jnp.concatenate([x, jnp.zeros((NP - N, x.shape[1]), F32)], axis=0)
    batch3 = jnp.concatenate(
        [batch, jnp.full((NP - N,), -1, I32)]).reshape(_NBR, 1, _BR)

    s1, s2 = _edge_scalars(edge_attr, U, c2)
    l1, l2 = _get_sc_loop_mean()(dst_r, s1.reshape(16, CGRP, 16),
                                 s2.reshape(16, CGRP, 16))
    se1 = jnp.concatenate([s1, l1[:N], pad_f]).reshape(16, NGRP, 16)
    se2 = jnp.concatenate([s2, l2[:N], pad_f]).reshape(16, NGRP, 16)
    se0 = jnp.concatenate([jnp.zeros((E2,), F32), pad_f]).reshape(16, NGRP, 16)

    srcga = src.reshape(16, CGRP, 16)
    dstga = dst.reshape(16, CGRP, 16)

    def _wsplit(wfull):
        wflat = wfull.reshape(E2P)
        wa = wflat[:E].reshape(16, CGRP, 16)
        wl3 = jnp.concatenate([wflat[E:E2], jnp.zeros((NP - N,), F32)])
        return wa, wl3.reshape(_NBR, 1, _BR)

    hw1, avs1, avd1 = _lin(x_pad, M1, m1b, as1, ad1)
    w_out1, den1 = _get_sc_edge_w()(avs1, avd1, srcg, dstg, se1)
    wa1, wl1 = _wsplit(w_out1)
    out1 = _get_sc_agg()(hw1, srcga, dstga, wa1, zer)

    hw2, avs2, avd2 = _lin_act(out1, hw1, wl1, den1, b1, W2, as2, ad2)
    w_out2, den2 = _get_sc_edge_w()(avs2, avd2, srcg, dstg, se2)
    wa2, wl2 = _wsplit(w_out2)
    out2 = _get_sc_agg()(hw2, srcga, dstga, wa2, zer)

    hwp1, avsp1, avdp1 = _lin_act(out2, hw2, wl2, den2, b2, Wp1, asp1, adp1)
    hwp2, avsp2, avdp2 = _lin_act(out2, hw2, wl2, den2, b2, Wp2, asp2, adp2)
    w_outp1, denp1 = _get_sc_edge_w()(avsp1, avdp1, srcg, dstg, se0)
    wap1, wlp1 = _wsplit(w_outp1)
    outp1 = _get_sc_agg()(hwp1, srcga, dstga, wap1, zer)
    sums1 = _pool(outp1, hwp1, wlp1, denp1, bp1, wproj1, bproj1, batch3)

    w_outp2, denp2 = _get_sc_edge_w()(avsp2, avdp2, srcg, dstg, se0)
    wap2, wlp2 = _wsplit(w_outp2)
    outp2 = _get_sc_agg()(hwp2, srcga, dstga, wap2, zer)
    sums2 = _pool(outp2, hwp2, wlp2, denp2, bp2, wproj2, bproj2, batch3)

    return _fin(sums1, sums2, batch3)


# speedup vs baseline: 1.0974x; 1.0001x over previous
import functools
import jax
import jax.numpy as jnp
from jax import lax
from jax.experimental import pallas as pl
from jax.experimental.pallas import tpu as pltpu
from jax.experimental.pallas import tpu_sc as plsc

F32 = jnp.float32
I32 = jnp.int32

N = 10000
E = 160000
G = 16
D_OUT = 512
NP = 10240
E2 = E + N
EPT = 10656
E2P = 16 * EPT
NGRP = EPT // 16
CPT = E // 16
CGRP = CPT // 16
BAND = NP // 16
NB4 = 4
FB = 128
RING = 4

@functools.lru_cache(maxsize=None)
def _get_mesh():
    return plsc.VectorSubcoreMesh(core_axis_name="c", subcore_axis_name="s")


def _sc_loop_mean_body(dst_hbm, s1_hbm, s2_hbm, l1_hbm, l2_hbm,
                       dst_v, s1_v, s2_v, cnt_v, su1_v, su2_v,
                       red_v, a_v, b_v, c_v, part):
    c = lax.axis_index("c")
    s = lax.axis_index("s")
    pltpu.sync_copy(dst_hbm.at[s], dst_v)
    pltpu.sync_copy(s1_hbm.at[s], s1_v)
    pltpu.sync_copy(s2_hbm.at[s], s2_v)

    zero16 = jnp.zeros((16,), F32)

    def zbody(i, _):
        cnt_v[pl.ds(i * 16, 16)] = zero16
        su1_v[pl.ds(i * 16, 16)] = zero16
        su2_v[pl.ds(i * 16, 16)] = zero16
        return 0
    lax.fori_loop(0, NP // 16, zbody, 0)

    one16 = jnp.full((16,), 1.0, F32)

    def ebody(g, _):
        dg = dst_v[g]
        plsc.addupdate_scatter(cnt_v, [dg], one16)
        plsc.addupdate_scatter(su1_v, [dg], s1_v[g])
        plsc.addupdate_scatter(su2_v, [dg], s2_v[g])
        return 0
    lax.fori_loop(0, CGRP, ebody, 0)

    pltpu.sync_copy(cnt_v, part.at[0, s])
    pltpu.sync_copy(su1_v, part.at[1, s])
    pltpu.sync_copy(su2_v, part.at[2, s])
    plsc.subcore_barrier()

    def _reduce(tab, outbuf):
        pltpu.sync_copy(part.at[tab, :, pl.ds(s * BAND, BAND)], red_v)

        def rbody(j, _):
            acc = jnp.zeros((16,), F32)
            for t in range(16):
                acc = acc + red_v[t, pl.ds(j * 16, 16)]
            outbuf[pl.ds(j * 16, 16)] = acc
            return 0
        lax.fori_loop(0, BAND // 16, rbody, 0)

    _reduce(0, c_v)
    _reduce(1, a_v)
    _reduce(2, b_v)

    def dbody(j, _):
        cc = jnp.maximum(c_v[pl.ds(j * 16, 16)], 1.0)
        a_v[pl.ds(j * 16, 16)] = a_v[pl.ds(j * 16, 16)] / cc
        b_v[pl.ds(j * 16, 16)] = b_v[pl.ds(j * 16, 16)] / cc
        return 0
    lax.fori_loop(0, BAND // 16, dbody, 0)

    @pl.when(c == 0)
    def _():
        pltpu.sync_copy(a_v, l1_hbm.at[pl.ds(s * BAND, BAND)])
        pltpu.sync_copy(b_v, l2_hbm.at[pl.ds(s * BAND, BAND)])


@functools.lru_cache(maxsize=None)
def _get_sc_loop_mean():
    return pl.kernel(
        _sc_loop_mean_body,
        out_type=(jax.ShapeDtypeStruct((NP,), F32),
                  jax.ShapeDtypeStruct((NP,), F32)),
        mesh=_get_mesh(),
        compiler_params=pltpu.CompilerParams(needs_layout_passes=False,
                                             use_tc_tiling_on_sc=False),
        scratch_types=[
        pltpu.VMEM((CGRP, 16), I32),
        pltpu.VMEM((CGRP, 16), F32),
        pltpu.VMEM((CGRP, 16), F32),
        pltpu.VMEM((NP,), F32),
        pltpu.VMEM((NP,), F32),
        pltpu.VMEM((NP,), F32),
        pltpu.VMEM((16, BAND), F32),
        pltpu.VMEM((BAND,), F32),
        pltpu.VMEM((BAND,), F32),
            pltpu.VMEM((BAND,), F32),
            pltpu.VMEM_SHARED((3, 16, NP), F32),
        ],
    )


def _sc_edge_w_body(avs_hbm, avd_hbm, src_hbm, dst_hbm, se_hbm,
                    w_hbm, den_hbm,
                    avs_v, avd_v, src_v, dst_v, se_v, w_v, den_v,
                    red_v, dout_v, denp):
    c = lax.axis_index("c")
    s = lax.axis_index("s")

    pltpu.sync_copy(avs_hbm, avs_v)
    pltpu.sync_copy(avd_hbm, avd_v)
    pltpu.sync_copy(src_hbm.at[s], src_v)
    pltpu.sync_copy(dst_hbm.at[s], dst_v)
    pltpu.sync_copy(se_hbm.at[s], se_v)

    zero16 = jnp.zeros((16,), F32)

    def zbody(i, _):
        den_v[pl.ds(i * 16, 16)] = zero16
        return 0
    lax.fori_loop(0, NP // 16, zbody, 0)

    def p1body(g, _):
        sg = src_v[g]
        dg = dst_v[g]
        a = (plsc.load_gather(avs_v, [sg]) + plsc.load_gather(avd_v, [dg])
             + se_v[g])
        a = jnp.where(a > 0, a, 0.2 * a)
        w = jnp.exp(a)
        w_v[g] = w
        plsc.addupdate_scatter(den_v, [dg], w)
        return 0
    lax.fori_loop(0, NGRP, p1body, 0)

    @pl.when(c == 0)
    def _():
        pltpu.sync_copy(w_v, w_hbm.at[s])

    pltpu.sync_copy(den_v, denp.at[s])
    plsc.subcore_barrier()

    pltpu.sync_copy(denp.at[:, pl.ds(s * BAND, BAND)], red_v)

    def rbody(j, _):
        acc = jnp.zeros((16,), F32)
        for t in range(16):
            acc = acc + red_v[t, pl.ds(j * 16, 16)]
        dout_v[pl.ds(j * 16, 16)] = acc
        return 0
    lax.fori_loop(0, BAND // 16, rbody, 0)

    @pl.when(c == 0)
    def _():
        pltpu.sync_copy(dout_v, den_hbm.at[pl.ds(s * BAND, BAND)])


@functools.lru_cache(maxsize=None)
def _get_sc_edge_w():
    return pl.kernel(
        _sc_edge_w_body,
        out_type=(jax.ShapeDtypeStruct((16, NGRP, 16), F32),
                  jax.ShapeDtypeStruct((NP,), F32)),
        mesh=_get_mesh(),
        compiler_params=pltpu.CompilerParams(needs_layout_passes=False,
                                             use_tc_tiling_on_sc=False),
        scratch_types=[
            pltpu.VMEM((NP,), F32),
            pltpu.VMEM((NP,), F32),
            pltpu.VMEM((NGRP, 16), I32),
            pltpu.VMEM((NGRP, 16), I32),
            pltpu.VMEM((NGRP, 16), F32),
            pltpu.VMEM((NGRP, 16), F32),
            pltpu.VMEM((NP,), F32),
            pltpu.VMEM((16, BAND), F32),
            pltpu.VMEM((BAND,), F32),
            pltpu.VMEM_SHARED((16, NP), F32),
        ],
    )


def _sc_agg_body(hw_hbm, src_hbm, dst_hbm, w_hbm, zer_hbm, out_hbm,
                 src_v, dst_v, w_v, gring, sring, gsem, ssem, table):
    c = lax.axis_index("c")
    s = lax.axis_index("s")

    pltpu.sync_copy(src_hbm.at[s], src_v)
    pltpu.sync_copy(dst_hbm.at[s], dst_v)
    pltpu.sync_copy(w_hbm.at[s], w_v)

    for bi in range(2):
        b = bi * 2 + c
        base = b * NP

        pltpu.sync_copy(zer_hbm, table.at[pl.ds(s * BAND, BAND)])
        plsc.subcore_barrier()

        def gstart(g, slot):
            idx = src_v[g] + base
            pltpu.async_copy(hw_hbm.at[idx], gring.at[slot], gsem.at[slot])

        for slot in range(RING):
            gstart(slot, slot)

        def mbody(g, _):
            slot = lax.rem(g, RING)
            gb = gring.at[slot]
            sb = sring.at[slot]
            pltpu.make_async_copy(hw_hbm.at[pl.ds(0, 16)], gb,
                                  gsem.at[slot]).wait()

            @pl.when(g >= RING)
            def _():
                pltpu.make_async_copy(sb, table.at[pl.ds(0, 16)],
                                      ssem.at[slot]).wait()

            wg = w_v[g]
            for r in range(16):
                wr = wg[r]
                for k in range(FB // 16):
                    sb[r, pl.ds(k * 16, 16)] = gb[r, pl.ds(k * 16, 16)] * wr

            @pl.when(g + RING < CGRP)
            def _():
                gstart(g + RING, slot)

            dg = dst_v[g]
            pltpu.async_copy(sb, table.at[dg], ssem.at[slot], add=True)
            return 0
        lax.fori_loop(0, CGRP, mbody, 0)

        for slot in range(RING):
            pltpu.make_async_copy(sring.at[slot], table.at[pl.ds(0, 16)],
                                  ssem.at[slot]).wait()
        plsc.subcore_barrier()

        pltpu.sync_copy(table.at[pl.ds(s * BAND, BAND)],
                        out_hbm.at[pl.ds(base + s * BAND, BAND)])
        plsc.subcore_barrier()


@functools.lru_cache(maxsize=None)
def _get_sc_agg():
    return pl.kernel(
        _sc_agg_body,
        out_type=jax.ShapeDtypeStruct((NB4 * NP, FB), F32),
        mesh=_get_mesh(),
        compiler_params=pltpu.CompilerParams(needs_layout_passes=False,
                                             use_tc_tiling_on_sc=False),
        scratch_types=[
            pltpu.VMEM((CGRP, 16), I32),
            pltpu.VMEM((CGRP, 16), I32),
            pltpu.VMEM((CGRP, 16), F32),
            pltpu.VMEM((RING, 16, FB), F32),
            pltpu.VMEM((RING, 16, FB), F32),
            pltpu.SemaphoreType.DMA((RING,)),
            pltpu.SemaphoreType.DMA((RING,)),
            pltpu.VMEM_SHARED((NP, FB), F32),
        ],
    )


_BR = 512
_NBR = NP // _BR


def _lin_body(h_ref, w_ref, b_ref, as_ref, ad_ref, hw_ref, avs_ref, avd_ref):
    b = pl.program_id(1)
    hwb = jnp.dot(h_ref[...], w_ref[...], preferred_element_type=F32)
    hwb = hwb + b_ref[...]
    hw_ref[...] = hwb
    pa = jnp.dot(hwb, as_ref[...].reshape(FB), preferred_element_type=F32)
    pd = jnp.dot(hwb, ad_ref[...].reshape(FB), preferred_element_type=F32)

    @pl.when(b == 0)
    def _():
        avs_ref[...] = jnp.zeros((1, 1, _BR), F32)
        avd_ref[...] = jnp.zeros((1, 1, _BR), F32)
    avs_ref[...] += pa.reshape(1, 1, _BR)
    avd_ref[...] += pd.reshape(1, 1, _BR)


def _lin(h, W, bias, asv, adv):
    K = h.shape[1]
    hw, avs, avd = pl.pallas_call(
        _lin_body,
        grid=(_NBR, NB4),
        in_specs=[
            pl.BlockSpec((_BR, K), lambda i, b: (i, 0)),
            pl.BlockSpec((K, FB), lambda i, b: (0, b)),
            pl.BlockSpec((1, FB), lambda i, b: (0, b)),
            pl.BlockSpec((1, FB), lambda i, b: (0, b)),
            pl.BlockSpec((1, FB), lambda i, b: (0, b)),
        ],
        out_specs=[
            pl.BlockSpec((_BR, FB), lambda i, b: (b * _NBR + i, 0)),
            pl.BlockSpec((1, 1, _BR), lambda i, b: (i, 0, 0)),
            pl.BlockSpec((1, 1, _BR), lambda i, b: (i, 0, 0)),
        ],
        out_shape=[
            jax.ShapeDtypeStruct((NB4 * NP, FB), F32),
            jax.ShapeDtypeStruct((_NBR, 1, _BR), F32),
            jax.ShapeDtypeStruct((_NBR, 1, _BR), F32),
        ],
    )(h, W, bias.reshape(1, D_OUT), asv.reshape(1, D_OUT),
      adv.reshape(1, D_OUT))
    return hw, avs.reshape(NP), avd.reshape(NP)


def _lin_act_body(o0, o1, o2, o3, h0, h1, h2, h3, wl_ref, den_ref,
                  bp_ref, w_ref, as_ref, ad_ref, hw_ref, avs_ref, avd_ref):
    b = pl.program_id(1)
    cat = jnp.concatenate([o0[...], o1[...], o2[...], o3[...]], axis=1)
    hwp = jnp.concatenate([h0[...], h1[...], h2[...], h3[...]], axis=1)
    wl = wl_ref[...].reshape(_BR, 1)
    den = den_ref[...].reshape(_BR, 1)
    hin = jnp.maximum((cat + wl * hwp) / (den + 1e-16) + bp_ref[...], 0.0)
    hwb = jnp.dot(hin, w_ref[...], preferred_element_type=F32)
    hw_ref[...] = hwb
    pa = jnp.dot(hwb, as_ref[...].reshape(FB), preferred_element_type=F32)
    pd = jnp.dot(hwb, ad_ref[...].reshape(FB), preferred_element_type=F32)

    @pl.when(b == 0)
    def _():
        avs_ref[...] = jnp.zeros((1, 1, _BR), F32)
        avd_ref[...] = jnp.zeros((1, 1, _BR), F32)
    avs_ref[...] += pa.reshape(1, 1, _BR)
    avd_ref[...] += pd.reshape(1, 1, _BR)


def _lin_act(outflat, hwflat, wl3, den, bprev, W, asv, adv):
    specs = [pl.BlockSpec((_BR, FB),
                          (lambda j: (lambda i, b: (j * _NBR + i, 0)))(j))
             for j in range(4)]
    hw, avs, avd = pl.pallas_call(
        _lin_act_body,
        grid=(_NBR, NB4),
        in_specs=specs + specs + [
            pl.BlockSpec((1, 1, _BR), lambda i, b: (i, 0, 0)),
            pl.BlockSpec((1, 1, _BR), lambda i, b: (i, 0, 0)),
            pl.BlockSpec((1, D_OUT), lambda i, b: (0, 0)),
            pl.BlockSpec((D_OUT, FB), lambda i, b: (0, b)),
            pl.BlockSpec((1, FB), lambda i, b: (0, b)),
            pl.BlockSpec((1, FB), lambda i, b: (0, b)),
        ],
        out_specs=[
            pl.BlockSpec((_BR, FB), lambda i, b: (b * _NBR + i, 0)),
            pl.BlockSpec((1, 1, _BR), lambda i, b: (i, 0, 0)),
            pl.BlockSpec((1, 1, _BR), lambda i, b: (i, 0, 0)),
        ],
        out_shape=[
            jax.ShapeDtypeStruct((NB4 * NP, FB), F32),
            jax.ShapeDtypeStruct((_NBR, 1, _BR), F32),
            jax.ShapeDtypeStruct((_NBR, 1, _BR), F32),
        ],
    )(outflat, outflat, outflat, outflat,
      hwflat, hwflat, hwflat, hwflat, wl3, den.reshape(_NBR, 1, _BR),
      bprev.reshape(1, D_OUT), W, asv.reshape(1, D_OUT),
      adv.reshape(1, D_OUT))
    return hw, avs.reshape(NP), avd.reshape(NP)


def _edge_scalar_body(ea_ref, u_ref, c_ref, s1_ref, s2_ref):
    sblk = jnp.dot(ea_ref[...], u_ref[...], preferred_element_type=F32)
    sblk = sblk + c_ref[...]
    s1_ref[...] = sblk[:, 0].reshape(1, 1, -1)
    s2_ref[...] = sblk[:, 1].reshape(1, 1, -1)


def _edge_scalars(edge_attr, U, c2):
    BE = 2000
    nb = E // BE
    D = edge_attr.shape[1]
    s1, s2 = pl.pallas_call(
        _edge_scalar_body,
        grid=(nb,),
        in_specs=[
            pl.BlockSpec((BE, D), lambda i: (i, 0)),
            pl.BlockSpec((D, 2), lambda i: (0, 0)),
            pl.BlockSpec((1, 2), lambda i: (0, 0)),
        ],
        out_specs=[
            pl.BlockSpec((1, 1, BE), lambda i: (i, 0, 0)),
            pl.BlockSpec((1, 1, BE), lambda i: (i, 0, 0)),
        ],
        out_shape=[
            jax.ShapeDtypeStruct((nb, 1, BE), F32),
            jax.ShapeDtypeStruct((nb, 1, BE), F32),
        ],
    )(edge_attr, U, c2.reshape(1, 2))
    return s1.reshape(E), s2.reshape(E)


def _pool_body(o0, o1, o2, o3, h0, h1, h2, h3, wl_ref, den_ref, b_ref,
               wp_ref, bp_ref, batch_ref, sums_ref):
    i = pl.program_id(0)
    cat = jnp.concatenate([o0[...], o1[...], o2[...], o3[...]], axis=1)
    hw = jnp.concatenate([h0[...], h1[...], h2[...], h3[...]], axis=1)
    wl = wl_ref[...].reshape(_BR, 1)
    den = den_ref[...].reshape(_BR, 1)
    hp = (cat + wl * hw) / (den + 1e-16) + b_ref[...]
    sc = jax.nn.sigmoid(jnp.dot(hp, wp_ref[...],
                                preferred_element_type=F32) + bp_ref[0, 0])
    xw = hp * sc
    bv = batch_ref[...].reshape(_BR, 1)
    oh = (bv == lax.broadcasted_iota(I32, (_BR, G), 1)).astype(F32)
    contrib = lax.dot_general(oh, xw, (((0,), (0,)), ((), ())),
                              preferred_element_type=F32)

    @pl.when(i == 0)
    def _():
        sums_ref[...] = jnp.zeros((G, D_OUT), F32)
    sums_ref[...] += contrib


def _pool(outflat, hwflat, wl3, den, bias, wproj, bproj, batch3):
    specs = [pl.BlockSpec((_BR, FB), (lambda j: (lambda i: (j * _NBR + i, 0)))(j))
             for j in range(4)]
    return pl.pallas_call(
        _pool_body,
        grid=(_NBR,),
        in_specs=specs + specs + [
            pl.BlockSpec((1, 1, _BR), lambda i: (i, 0, 0)),
            pl.BlockSpec((1, 1, _BR), lambda i: (i, 0, 0)),
            pl.BlockSpec((1, D_OUT), lambda i: (0, 0)),
            pl.BlockSpec((D_OUT, 1), lambda i: (0, 0)),
            pl.BlockSpec((1, 1), lambda i: (0, 0)),
            pl.BlockSpec((1, 1, _BR), lambda i: (i, 0, 0)),
        ],
        out_specs=pl.BlockSpec((G, D_OUT), lambda i: (0, 0)),
        out_shape=jax.ShapeDtypeStruct((G, D_OUT), F32),
    )(outflat, outflat, outflat, outflat,
      hwflat, hwflat, hwflat, hwflat, wl3,
      den.reshape(_NBR, 1, _BR),
      bias.reshape(1, D_OUT), wproj, bproj.reshape(1, 1), batch3)


def _fin_body(s1_ref, s2_ref, batch_ref, out_ref):
    bm = batch_ref[...].reshape(_NBR, _BR)
    acc = jnp.zeros((G,), F32)
    for r in range(_NBR):
        oh = (bm[r].reshape(_BR, 1)
              == lax.broadcasted_iota(I32, (_BR, G), 1)).astype(F32)
        acc = acc + jnp.sum(oh, axis=0)
    cnt = jnp.maximum(acc, 1.0).reshape(G, 1)
    out_ref[...] = jnp.concatenate(
        [s1_ref[...] / cnt, s2_ref[...] / cnt], axis=1)


def _fin(sums1, sums2, batch3):
    return pl.pallas_call(
        _fin_body,
        grid=(1,),
        in_specs=[
            pl.BlockSpec((G, D_OUT), lambda i: (0, 0)),
            pl.BlockSpec((G, D_OUT), lambda i: (0, 0)),
            pl.BlockSpec((_NBR, 1, _BR), lambda i: (0, 0, 0)),
        ],
        out_specs=pl.BlockSpec((G, 2 * D_OUT), lambda i: (0, 0)),
        out_shape=jax.ShapeDtypeStruct((G, 2 * D_OUT), F32),
    )(sums1, sums2, batch3)


def kernel(x, edge_index, edge_attr, batch, Wnp, bnp, Wep, bep,
           W1, as1, ad1, b1, We1, ae1, W2, as2, ad2, b2, We2, ae2,
           Wp1, asp1, adp1, bp1, wproj1, bproj1,
           Wp2, asp2, adp2, bp2, wproj2, bproj2):
    src = edge_index[0]
    dst = edge_index[1]

    v1 = We1 @ ae1
    v2 = We2 @ ae2
    U = jnp.stack([Wep @ v1, Wep @ v2], axis=1)
    c2 = jnp.stack([bep @ v1, bep @ v2])
    M1 = Wnp @ W1
    m1b = bnp @ W1

    loop_ids = jnp.arange(N, dtype=I32)
    pad_i = jnp.zeros((E2P - E2,), I32)
    srcg = jnp.concatenate([src, loop_ids, pad_i]).reshape(16, NGRP, 16)
    dstg = jnp.concatenate([dst, loop_ids, pad_i]).reshape(16, NGRP, 16)
    pad_f = jnp.full((E2P - E2,), -1e30, F32)
    dst_r = dst.reshape(16, CGRP, 16)
    zer = jnp.zeros((BAND, FB), F32)
    x_pad = jnp.concatenate([x, jnp.zeros((NP - N, x.shape[1]), F32)], axis=0)
    batch3 = jnp.concatenate(
        [batch, jnp.full((NP - N,), -1, I32)]).reshape(_NBR, 1, _BR)

    s1, s2 = _edge_scalars(edge_attr, U, c2)
    l1, l2 = _get_sc_loop_mean()(dst_r, s1.reshape(16, CGRP, 16),
                                 s2.reshape(16, CGRP, 16))
    se1 = jnp.concatenate([s1, l1[:N], pad_f]).reshape(16, NGRP, 16)
    se2 = jnp.concatenate([s2, l2[:N], pad_f]).reshape(16, NGRP, 16)
    se0 = jnp.concatenate([jnp.zeros((E2,), F32), pad_f]).reshape(16, NGRP, 16)

    srcga = src.reshape(16, CGRP, 16)
    dstga = dst.reshape(16, CGRP, 16)

    def _wsplit(wfull):
        wflat = wfull.reshape(E2P)
        wa = wflat[:E].reshape(16, CGRP, 16)
        wl3 = jnp.concatenate([wflat[E:E2], jnp.zeros((NP - N,), F32)])
        return wa, wl3.reshape(_NBR, 1, _BR)

    hw1, avs1, avd1 = _lin(x_pad, M1, m1b, as1, ad1)
    w_out1, den1 = _get_sc_edge_w()(avs1, avd1, srcg, dstg, se1)
    wa1, wl1 = _wsplit(w_out1)
    out1 = _get_sc_agg()(hw1, srcga, dstga, wa1, zer)

    hw2, avs2, avd2 = _lin_act(out1, hw1, wl1, den1, b1, W2, as2, ad2)
    w_out2, den2 = _get_sc_edge_w()(avs2, avd2, srcg, dstg, se2)
    wa2, wl2 = _wsplit(w_out2)
    out2 = _get_sc_agg()(hw2, srcga, dstga, wa2, zer)

    hwp1, avsp1, avdp1 = _lin_act(out2, hw2, wl2, den2, b2, Wp1, asp1, adp1)
    hwp2, avsp2, avdp2 = _lin_act(out2, hw2, wl2, den2, b2, Wp2, asp2, adp2)
    w_outp1, denp1 = _get_sc_edge_w()(avsp1, avdp1, srcg, dstg, se0)
    wap1, wlp1 = _wsplit(w_outp1)
    outp1 = _get_sc_agg()(hwp1, srcga, dstga, wap1, zer)
    sums1 = _pool(outp1, hwp1, wlp1, denp1, bp1, wproj1, bproj1, batch3)

    w_outp2, denp2 = _get_sc_edge_w()(avsp2, avdp2, srcg, dstg, se0)
    wap2, wlp2 = _wsplit(w_outp2)
    outp2 = _get_sc_agg()(hwp2, srcga, dstga, wap2, zer)
    sums2 = _pool(outp2, hwp2, wlp2, denp2, bp2, wproj2, bproj2, batch3)

    return _fin(sums1, sums2, batch3)
